# Initial kernel scaffold; baseline (speedup 1.0000x reference)
import functools

import jax
import jax.numpy as jnp
from jax import lax
from jax.experimental import pallas as pl
from jax.experimental.pallas import tpu as pltpu
from jax.experimental.pallas import tpu_sc as plsc

NC = 2
NS = 16
BN = 1000
KE = 2000
CK = 80


def _stage_a_body(x_ref, pos_ref, W_in_ref, b_in_ref, W_src_ref, W_dst_ref,
                  W_lin_ref, Wp1_ref, bp1_ref, Wp2_ref, bp2_ref,
                  Wa1_ref, ba1_ref, Wa2_ref, ba2_ref,
                  h_ref, p1_ref, sL_ref, alphaL_ref, cmax_ref):
    pid = pl.program_id(0)
    h = jnp.maximum(x_ref[...] @ W_in_ref[...] + b_in_ref[...], 0.0)
    h_ref[...] = h
    p1_ref[...] = pos_ref[...] @ Wp1_ref[...]
    d0 = jnp.maximum(jnp.maximum(bp1_ref[...], 0.0) @ Wp2_ref[...]
                     + bp2_ref[...], 0.0)
    sL_ref[...] = h @ W_lin_ref[...] + d0
    t = h @ W_dst_ref[...] - h @ W_src_ref[...] + d0
    a1 = jnp.maximum(t @ Wa1_ref[...] + ba1_ref[...], 0.0)
    aL = jnp.maximum(a1 @ Wa2_ref[...] + ba2_ref[...], 0.0)
    alphaL_ref[...] = aL

    @pl.when(pid == 0)
    def _():
        cmax_ref[...] = jnp.zeros_like(cmax_ref)

    cm = jnp.max(aL, axis=0, keepdims=True)
    cmax_ref[...] = jnp.maximum(cmax_ref[...], jnp.broadcast_to(cm, cmax_ref.shape))


def _stage_a(n, x, pos8, W_in, b_in, W_src, W_dst, W_lin, Wp1_8, bp1, Wp2, bp2,
             Wa1, ba1, Wa2, ba2):
    D = x.shape[1]
    H = Wp1_8.shape[1]
    grid = (n // BN,)
    full = lambda shape: pl.BlockSpec(shape, lambda i: (0, 0))
    row = lambda w: pl.BlockSpec((BN, w), lambda i: (i, 0))
    return pl.pallas_call(
        _stage_a_body,
        grid=grid,
        in_specs=[row(D), pl.BlockSpec((BN, 8), lambda i: (i, 0)),
                  full((D, D)), full((1, D)), full((D, D)), full((D, D)),
                  full((D, D)), full((8, H)), full((1, H)), full((H, D)),
                  full((1, D)), full((D, H)), full((1, H)), full((H, D)),
                  full((1, D))],
        out_specs=[row(D), row(H), row(D), row(D),
                   pl.BlockSpec((8, D), lambda i: (0, 0))],
        out_shape=[jax.ShapeDtypeStruct((n, D), jnp.float32),
                   jax.ShapeDtypeStruct((n, H), jnp.float32),
                   jax.ShapeDtypeStruct((n, D), jnp.float32),
                   jax.ShapeDtypeStruct((n, D), jnp.float32),
                   jax.ShapeDtypeStruct((8, D), jnp.float32)],
    )(x, pos8, W_in, b_in, W_src, W_dst, W_lin, Wp1_8, bp1, Wp2, bp2,
      Wa1, ba1, Wa2, ba2)


def _gather_sc(h, p1, ii, jj):
    n, D = h.shape
    H = p1.shape[1]
    E = ii.shape[0]
    EW = E // (NC * NS)
    NCH = EW // CK

    @functools.partial(
        pl.kernel,
        out_type=[jax.ShapeDtypeStruct((E, D), jnp.float32),
                  jax.ShapeDtypeStruct((E, D), jnp.float32),
                  jax.ShapeDtypeStruct((E, H), jnp.float32),
                  jax.ShapeDtypeStruct((E, H), jnp.float32)],
        mesh=plsc.VectorSubcoreMesh(core_axis_name="c", subcore_axis_name="s"),
        scratch_types=[pltpu.VMEM((CK,), jnp.int32),
                       pltpu.VMEM((CK,), jnp.int32),
                       pltpu.VMEM((CK, D), jnp.float32),
                       pltpu.VMEM((CK, D), jnp.float32),
                       pltpu.VMEM((CK, H), jnp.float32),
                       pltpu.VMEM((CK, H), jnp.float32),
                       pltpu.SemaphoreType.DMA],
    )
    def k(h_hbm, p1_hbm, ii_hbm, jj_hbm, hi_hbm, hj_hbm, qi_hbm, qj_hbm,
          ii_v, jj_v, hi_v, hj_v, qi_v, qj_v, sem):
        wid = lax.axis_index("s") * NC + lax.axis_index("c")
        base = wid * EW

        def chunk(kk, carry):
            e0 = base + kk * CK
            pltpu.sync_copy(ii_hbm.at[pl.ds(e0, CK)], ii_v)
            pltpu.sync_copy(jj_hbm.at[pl.ds(e0, CK)], jj_v)
            c1 = pltpu.async_copy(h_hbm.at[ii_v], hi_v, sem)
            c2 = pltpu.async_copy(h_hbm.at[jj_v], hj_v, sem)
            c3 = pltpu.async_copy(p1_hbm.at[ii_v], qi_v, sem)
            c4 = pltpu.async_copy(p1_hbm.at[jj_v], qj_v, sem)
            c1.wait(); c2.wait(); c3.wait(); c4.wait()
            pltpu.sync_copy(hi_v, hi_hbm.at[pl.ds(e0, CK)])
            pltpu.sync_copy(hj_v, hj_hbm.at[pl.ds(e0, CK)])
            pltpu.sync_copy(qi_v, qi_hbm.at[pl.ds(e0, CK)])
            pltpu.sync_copy(qj_v, qj_hbm.at[pl.ds(e0, CK)])
            return carry

        lax.fori_loop(0, NCH, chunk, 0)

    return k(h, p1, ii, jj)


def _stage_c_body(hi_ref, hj_ref, qi_ref, qj_ref, W_src_ref, W_dst_ref,
                  W_lin_ref, bp1_ref, Wp2_ref, bp2_ref, Wa1_ref, ba1_ref,
                  Wa2_ref, ba2_ref, alpha_ref, s_ref, cmax_ref):
    pid = pl.program_id(0)
    hi = hi_ref[...]
    hj = hj_ref[...]
    g1 = qi_ref[...] - qj_ref[...] + bp1_ref[...]
    delta = jnp.maximum(jnp.maximum(g1, 0.0) @ Wp2_ref[...] + bp2_ref[...], 0.0)
    t = hi @ W_dst_ref[...] - hj @ W_src_ref[...] + delta
    a1 = jnp.maximum(t @ Wa1_ref[...] + ba1_ref[...], 0.0)
    alpha = jnp.maximum(a1 @ Wa2_ref[...] + ba2_ref[...], 0.0)
    alpha_ref[...] = alpha
    s_ref[...] = hj @ W_lin_ref[...] + delta

    @pl.when(pid == 0)
    def _():
        cmax_ref[...] = jnp.zeros_like(cmax_ref)

    cm = jnp.max(alpha, axis=0, keepdims=True)
    cmax_ref[...] = jnp.maximum(cmax_ref[...], jnp.broadcast_to(cm, cmax_ref.shape))


def _stage_c(hi, hj, qi, qj, W_src, W_dst, W_lin, bp1, Wp2, bp2, Wa1, ba1,
             Wa2, ba2):
    E, D = hi.shape
    H = qi.shape[1]
    grid = (E // KE,)
    full = lambda shape: pl.BlockSpec(shape, lambda i: (0, 0))
    row = lambda w: pl.BlockSpec((KE, w), lambda i: (i, 0))
    return pl.pallas_call(
        _stage_c_body,
        grid=grid,
        in_specs=[row(D), row(D), row(H), row(H),
                  full((D, D)), full((D, D)), full((D, D)), full((1, H)),
                  full((H, D)), full((1, D)), full((D, H)), full((1, H)),
                  full((H, D)), full((1, D))],
        out_specs=[row(D), row(D), pl.BlockSpec((8, D), lambda i: (0, 0))],
        out_shape=[jax.ShapeDtypeStruct((E, D), jnp.float32),
                   jax.ShapeDtypeStruct((E, D), jnp.float32),
                   jax.ShapeDtypeStruct((8, D), jnp.float32)],
    )(hi, hj, qi, qj, W_src, W_dst, W_lin, bp1, Wp2, bp2, Wa1, ba1, Wa2, ba2)


def _scatter_sc(alpha, s_arr, ii, cmax, zero):
    E, D = alpha.shape
    n = zero.shape[0]
    Hh = D // 2
    ESC = E // NS
    NCH = ESC // CK
    NPS = n // NS

    @functools.partial(
        pl.kernel,
        out_type=jax.ShapeDtypeStruct((NC, n, D), jnp.float32),
        mesh=plsc.VectorSubcoreMesh(core_axis_name="c", subcore_axis_name="s"),
        scratch_types=[pltpu.VMEM((CK,), jnp.int32),
                       pltpu.VMEM((CK, Hh), jnp.float32),
                       pltpu.VMEM((CK, Hh), jnp.float32),
                       pltpu.VMEM((CK, D), jnp.float32),
                       pltpu.VMEM((Hh,), jnp.float32),
                       pltpu.VMEM_SHARED((n, D), jnp.float32)],
    )
    def k(alpha_hbm, s_hbm, ii_hbm, cmax_hbm, zero_hbm, out_hbm,
          idx_v, a_v, s_v, pay_v, cm_v, acc_sh):
        c = lax.axis_index("c")
        sid = lax.axis_index("s")
        pltpu.sync_copy(zero_hbm.at[pl.ds(sid * NPS, NPS)],
                        acc_sh.at[pl.ds(sid * NPS, NPS)])
        pltpu.sync_copy(cmax_hbm.at[pl.ds(c * Hh, Hh)], cm_v)
        plsc.subcore_barrier()
        cms = [cm_v[pl.ds(g * 16, 16)] for g in range(Hh // 16)]

        def chunk(kk, carry):
            e0 = sid * ESC + kk * CK
            pltpu.sync_copy(ii_hbm.at[pl.ds(e0, CK)], idx_v)
            pltpu.sync_copy(alpha_hbm.at[pl.ds(e0, CK), pl.ds(c * Hh, Hh)], a_v)
            pltpu.sync_copy(s_hbm.at[pl.ds(e0, CK), pl.ds(c * Hh, Hh)], s_v)

            def rowfn(r, carry2):
                for g in range(Hh // 16):
                    ex = jnp.exp(a_v[r, pl.ds(g * 16, 16)] - cms[g])
                    pay_v[r, pl.ds(Hh + g * 16, 16)] = ex
                    pay_v[r, pl.ds(g * 16, 16)] = ex * s_v[r, pl.ds(g * 16, 16)]
                return carry2

            lax.fori_loop(0, CK, rowfn, 0)
            pltpu.sync_copy(pay_v, acc_sh.at[idx_v], add=True)
            return carry

        lax.fori_loop(0, NCH, chunk, 0)
        plsc.subcore_barrier()
        pltpu.sync_copy(acc_sh.at[pl.ds(sid * NPS, NPS)],
                        out_hbm.at[c, pl.ds(sid * NPS, NPS)])

    return k(alpha, s_arr, ii, cmax, zero)


def _stage_e_body(acc_ref, alphaL_ref, sL_ref, cmax_ref, W_out_ref, b_out_ref,
                  o_ref):
    D = o_ref.shape[1]
    Hh = D // 2
    exL = jnp.exp(alphaL_ref[...] - cmax_ref[...])
    sL = sL_ref[...]
    num0 = acc_ref[0, :, 0:Hh] + exL[:, 0:Hh] * sL[:, 0:Hh]
    den0 = acc_ref[0, :, Hh:D] + exL[:, 0:Hh]
    num1 = acc_ref[1, :, 0:Hh] + exL[:, Hh:D] * sL[:, Hh:D]
    den1 = acc_ref[1, :, Hh:D] + exL[:, Hh:D]
    o0 = num0 / (den0 + 1e-16)
    o1 = num1 / (den1 + 1e-16)
    out = (o0 @ W_out_ref[0:Hh, :] + o1 @ W_out_ref[Hh:D, :]) + b_out_ref[...]
    o_ref[...] = jnp.maximum(out, 0.0)


def _stage_e(accsc, alphaL, sL, cmax_row, W_out, b_out):
    n, D = alphaL.shape
    grid = (n // BN,)
    full = lambda shape: pl.BlockSpec(shape, lambda i: (0, 0))
    row = lambda w: pl.BlockSpec((BN, w), lambda i: (i, 0))
    return pl.pallas_call(
        _stage_e_body,
        grid=grid,
        in_specs=[pl.BlockSpec((NC, BN, D), lambda i: (0, i, 0)),
                  row(D), row(D), full((1, D)), full((D, D)), full((1, D))],
        out_specs=row(D),
        out_shape=jax.ShapeDtypeStruct((n, D), jnp.float32),
    )(accsc, alphaL, sL, cmax_row, W_out, b_out)


def kernel(x, pos, edge_index, W_in, b_in, W_out, b_out, W_lin, W_src, W_dst,
           Wp1, bp1, Wp2, bp2, Wa1, ba1, Wa2, ba2):
    n, D = x.shape
    jj = edge_index[0].astype(jnp.int32)
    ii = edge_index[1].astype(jnp.int32)
    pos8 = jnp.pad(pos.astype(jnp.float32), ((0, 0), (0, 8 - pos.shape[1])))
    Wp1_8 = jnp.pad(Wp1, ((0, 8 - Wp1.shape[0]), (0, 0)))
    r1 = lambda v: v.reshape(1, -1)

    h, p1, sL, alphaL, cmaxA = _stage_a(
        n, x, pos8, W_in, r1(b_in), W_src, W_dst, W_lin, Wp1_8, r1(bp1), Wp2,
        r1(bp2), Wa1, r1(ba1), Wa2, r1(ba2))
    hi, hj, qi, qj = _gather_sc(h, p1, ii, jj)
    alpha, s_arr, cmaxC = _stage_c(
        hi, hj, qi, qj, W_src, W_dst, W_lin, r1(bp1), Wp2, r1(bp2), Wa1,
        r1(ba1), Wa2, r1(ba2))
    cmax = jnp.max(jnp.maximum(cmaxA, cmaxC), axis=0)
    zero = jnp.zeros((n, D), jnp.float32)
    accsc = _scatter_sc(alpha, s_arr, ii, cmax, zero)
    return _stage_e(accsc, alphaL, sL, cmax.reshape(1, -1), W_out, b_out)


# speedup vs baseline: 4.8160x; 4.8160x over previous
import functools

import jax
import jax.numpy as jnp
from jax import lax
from jax.experimental import pallas as pl
from jax.experimental.pallas import tpu as pltpu
from jax.experimental.pallas import tpu_sc as plsc

NC = 2
NS = 16
BN = 1000
KE = 2000
CK = 80


def _stage_a_body(x_ref, pos_ref, W_in_ref, b_in_ref, W_src_ref, W_dst_ref,
                  W_lin_ref, Wp1_ref, bp1_ref, Wp2_ref, bp2_ref,
                  Wa1_ref, ba1_ref, Wa2_ref, ba2_ref,
                  h_ref, p1_ref, sL_ref, alphaL_ref, cmax_ref):
    pid = pl.program_id(0)
    h = jnp.maximum(x_ref[...] @ W_in_ref[...] + b_in_ref[...], 0.0)
    h_ref[...] = h
    p1_ref[...] = pos_ref[...] @ Wp1_ref[...]
    d0 = jnp.maximum(jnp.maximum(bp1_ref[...], 0.0) @ Wp2_ref[...]
                     + bp2_ref[...], 0.0)
    sL_ref[...] = h @ W_lin_ref[...] + d0
    t = h @ W_dst_ref[...] - h @ W_src_ref[...] + d0
    a1 = jnp.maximum(t @ Wa1_ref[...] + ba1_ref[...], 0.0)
    aL = jnp.maximum(a1 @ Wa2_ref[...] + ba2_ref[...], 0.0)
    alphaL_ref[...] = aL

    @pl.when(pid == 0)
    def _():
        cmax_ref[...] = jnp.zeros_like(cmax_ref)

    cm = jnp.max(aL, axis=0, keepdims=True)
    cmax_ref[...] = jnp.maximum(cmax_ref[...], jnp.broadcast_to(cm, cmax_ref.shape))


def _stage_a(n, x, pos8, W_in, b_in, W_src, W_dst, W_lin, Wp1_8, bp1, Wp2, bp2,
             Wa1, ba1, Wa2, ba2):
    D = x.shape[1]
    H = Wp1_8.shape[1]
    grid = (n // BN,)
    full = lambda shape: pl.BlockSpec(shape, lambda i: (0, 0))
    row = lambda w: pl.BlockSpec((BN, w), lambda i: (i, 0))
    return pl.pallas_call(
        _stage_a_body,
        grid=grid,
        in_specs=[row(D), pl.BlockSpec((BN, 8), lambda i: (i, 0)),
                  full((D, D)), full((1, D)), full((D, D)), full((D, D)),
                  full((D, D)), full((8, H)), full((1, H)), full((H, D)),
                  full((1, D)), full((D, H)), full((1, H)), full((H, D)),
                  full((1, D))],
        out_specs=[row(D), row(H), row(D), row(D),
                   pl.BlockSpec((8, D), lambda i: (0, 0))],
        out_shape=[jax.ShapeDtypeStruct((n, D), jnp.float32),
                   jax.ShapeDtypeStruct((n, H), jnp.float32),
                   jax.ShapeDtypeStruct((n, D), jnp.float32),
                   jax.ShapeDtypeStruct((n, D), jnp.float32),
                   jax.ShapeDtypeStruct((8, D), jnp.float32)],
    )(x, pos8, W_in, b_in, W_src, W_dst, W_lin, Wp1_8, bp1, Wp2, bp2,
      Wa1, ba1, Wa2, ba2)


def _gather_sc(h, p1, ii, jj):
    n, D = h.shape
    H = p1.shape[1]
    E = ii.shape[0]
    EW = E // (NC * NS)
    NCH = EW // CK

    @functools.partial(
        pl.kernel,
        out_type=[jax.ShapeDtypeStruct((E, D), jnp.float32),
                  jax.ShapeDtypeStruct((E, D), jnp.float32),
                  jax.ShapeDtypeStruct((E, H), jnp.float32),
                  jax.ShapeDtypeStruct((E, H), jnp.float32)],
        mesh=plsc.VectorSubcoreMesh(core_axis_name="c", subcore_axis_name="s"),
        scratch_types=[pltpu.VMEM((CK,), jnp.int32),
                       pltpu.VMEM((CK,), jnp.int32),
                       pltpu.VMEM((CK, D), jnp.float32),
                       pltpu.VMEM((CK, D), jnp.float32),
                       pltpu.VMEM((CK, H), jnp.float32),
                       pltpu.VMEM((CK, H), jnp.float32),
                       pltpu.SemaphoreType.DMA],
    )
    def k(h_hbm, p1_hbm, ii_hbm, jj_hbm, hi_hbm, hj_hbm, qi_hbm, qj_hbm,
          ii_v, jj_v, hi_v, hj_v, qi_v, qj_v, sem):
        wid = lax.axis_index("s") * NC + lax.axis_index("c")
        base = wid * EW

        def chunk(kk, carry):
            e0 = base + kk * CK
            pltpu.sync_copy(ii_hbm.at[pl.ds(e0, CK)], ii_v)
            pltpu.sync_copy(jj_hbm.at[pl.ds(e0, CK)], jj_v)
            c1 = pltpu.async_copy(h_hbm.at[ii_v], hi_v, sem)
            c2 = pltpu.async_copy(h_hbm.at[jj_v], hj_v, sem)
            c3 = pltpu.async_copy(p1_hbm.at[ii_v], qi_v, sem)
            c4 = pltpu.async_copy(p1_hbm.at[jj_v], qj_v, sem)
            c1.wait(); c2.wait(); c3.wait(); c4.wait()
            pltpu.sync_copy(hi_v, hi_hbm.at[pl.ds(e0, CK)])
            pltpu.sync_copy(hj_v, hj_hbm.at[pl.ds(e0, CK)])
            pltpu.sync_copy(qi_v, qi_hbm.at[pl.ds(e0, CK)])
            pltpu.sync_copy(qj_v, qj_hbm.at[pl.ds(e0, CK)])
            return carry

        lax.fori_loop(0, NCH, chunk, 0)

    return k(h, p1, ii, jj)


def _stage_c_body(hi_ref, hj_ref, qi_ref, qj_ref, W_src_ref, W_dst_ref,
                  W_lin_ref, bp1_ref, Wp2_ref, bp2_ref, Wa1_ref, ba1_ref,
                  Wa2_ref, ba2_ref, alpha_ref, s_ref, cmax_ref):
    pid = pl.program_id(0)
    hi = hi_ref[...]
    hj = hj_ref[...]
    Hp = bp1_ref.shape[1]
    g1 = qi_ref[:, 0:Hp] - qj_ref[:, 0:Hp] + bp1_ref[...]
    delta = jnp.maximum(jnp.maximum(g1, 0.0) @ Wp2_ref[...] + bp2_ref[...], 0.0)
    t = hi @ W_dst_ref[...] - hj @ W_src_ref[...] + delta
    a1 = jnp.maximum(t @ Wa1_ref[...] + ba1_ref[...], 0.0)
    alpha = jnp.maximum(a1 @ Wa2_ref[...] + ba2_ref[...], 0.0)
    s = hj @ W_lin_ref[...] + delta
    Dh = alpha.shape[1] // 2
    alpha_ref[0] = alpha[:, 0:Dh]
    alpha_ref[1] = alpha[:, Dh:]
    s_ref[0] = s[:, 0:Dh]
    s_ref[1] = s[:, Dh:]

    @pl.when(pid == 0)
    def _():
        cmax_ref[...] = jnp.zeros_like(cmax_ref)

    cm = jnp.max(alpha, axis=0, keepdims=True)
    cmax_ref[...] = jnp.maximum(cmax_ref[...], jnp.broadcast_to(cm, cmax_ref.shape))


def _stage_c(hi, hj, qi, qj, W_src, W_dst, W_lin, bp1, Wp2, bp2, Wa1, ba1,
             Wa2, ba2):
    E, D = hi.shape
    H = bp1.shape[1]
    grid = (E // KE,)
    full = lambda shape: pl.BlockSpec(shape, lambda i: (0, 0))
    row = lambda w: pl.BlockSpec((KE, w), lambda i: (i, 0))
    return pl.pallas_call(
        _stage_c_body,
        grid=grid,
        in_specs=[row(D), row(D), row(qi.shape[1]), row(qj.shape[1]),
                  full((D, D)), full((D, D)), full((D, D)), full((1, H)),
                  full((H, D)), full((1, D)), full((D, H)), full((1, H)),
                  full((H, D)), full((1, D))],
        out_specs=[pl.BlockSpec((NC, KE, D // 2), lambda i: (0, i, 0)),
                   pl.BlockSpec((NC, KE, D // 2), lambda i: (0, i, 0)),
                   pl.BlockSpec((8, D), lambda i: (0, 0))],
        out_shape=[jax.ShapeDtypeStruct((NC, E, D // 2), jnp.float32),
                   jax.ShapeDtypeStruct((NC, E, D // 2), jnp.float32),
                   jax.ShapeDtypeStruct((8, D), jnp.float32)],
    )(hi, hj, qi, qj, W_src, W_dst, W_lin, bp1, Wp2, bp2, Wa1, ba1, Wa2, ba2)


def _scatter_sc(alpha, s_arr, ii, cmax, zero):
    _, E, Hh = alpha.shape
    D = 2 * Hh
    n = zero.shape[0]
    ESC = E // NS
    NCH = ESC // CK
    NPS = n // NS

    @functools.partial(
        pl.kernel,
        out_type=jax.ShapeDtypeStruct((NC, n, D), jnp.float32),
        mesh=plsc.VectorSubcoreMesh(core_axis_name="c", subcore_axis_name="s"),
        scratch_types=[pltpu.VMEM((CK,), jnp.int32),
                       pltpu.VMEM((CK, Hh), jnp.float32),
                       pltpu.VMEM((CK, Hh), jnp.float32),
                       pltpu.VMEM((CK, D), jnp.float32),
                       pltpu.VMEM((Hh,), jnp.float32),
                       pltpu.VMEM_SHARED((n, D), jnp.float32)],
    )
    def k(alpha_hbm, s_hbm, ii_hbm, cmax_hbm, zero_hbm, out_hbm,
          idx_v, a_v, s_v, pay_v, cm_v, acc_sh):
        c = lax.axis_index("c")
        sid = lax.axis_index("s")
        pltpu.sync_copy(zero_hbm.at[pl.ds(sid * NPS, NPS)],
                        acc_sh.at[pl.ds(sid * NPS, NPS)])
        pltpu.sync_copy(cmax_hbm.at[c], cm_v)
        plsc.subcore_barrier()
        cms = [cm_v[pl.ds(g * 16, 16)] for g in range(Hh // 16)]

        def chunk(kk, carry):
            e0 = sid * ESC + kk * CK
            pltpu.sync_copy(ii_hbm.at[pl.ds(e0, CK)], idx_v)
            pltpu.sync_copy(alpha_hbm.at[c, pl.ds(e0, CK)], a_v)
            pltpu.sync_copy(s_hbm.at[c, pl.ds(e0, CK)], s_v)

            def rowfn(r, carry2):
                for g in range(Hh // 16):
                    ex = jnp.exp(a_v[r, pl.ds(g * 16, 16)] - cms[g])
                    pay_v[r, pl.ds(Hh + g * 16, 16)] = ex
                    pay_v[r, pl.ds(g * 16, 16)] = ex * s_v[r, pl.ds(g * 16, 16)]
                return carry2

            lax.fori_loop(0, CK, rowfn, 0)
            pltpu.sync_copy(pay_v, acc_sh.at[idx_v], add=True)
            return carry

        lax.fori_loop(0, NCH, chunk, 0)
        plsc.subcore_barrier()
        pltpu.sync_copy(acc_sh.at[pl.ds(sid * NPS, NPS)],
                        out_hbm.at[c, pl.ds(sid * NPS, NPS)])

    return k(alpha, s_arr, ii, cmax, zero)


def _stage_e_body(acc_ref, alphaL_ref, sL_ref, cmax_ref, W_out_ref, b_out_ref,
                  o_ref):
    D = o_ref.shape[1]
    Hh = D // 2
    exL = jnp.exp(alphaL_ref[...] - cmax_ref[...])
    sL = sL_ref[...]
    num0 = acc_ref[0, :, 0:Hh] + exL[:, 0:Hh] * sL[:, 0:Hh]
    den0 = acc_ref[0, :, Hh:D] + exL[:, 0:Hh]
    num1 = acc_ref[1, :, 0:Hh] + exL[:, Hh:D] * sL[:, Hh:D]
    den1 = acc_ref[1, :, Hh:D] + exL[:, Hh:D]
    o0 = num0 / (den0 + 1e-16)
    o1 = num1 / (den1 + 1e-16)
    out = (o0 @ W_out_ref[0:Hh, :] + o1 @ W_out_ref[Hh:D, :]) + b_out_ref[...]
    o_ref[...] = jnp.maximum(out, 0.0)


def _stage_e(accsc, alphaL, sL, cmax_row, W_out, b_out):
    n, D = alphaL.shape
    grid = (n // BN,)
    full = lambda shape: pl.BlockSpec(shape, lambda i: (0, 0))
    row = lambda w: pl.BlockSpec((BN, w), lambda i: (i, 0))
    return pl.pallas_call(
        _stage_e_body,
        grid=grid,
        in_specs=[pl.BlockSpec((NC, BN, D), lambda i: (0, i, 0)),
                  row(D), row(D), full((1, D)), full((D, D)), full((1, D))],
        out_specs=row(D),
        out_shape=jax.ShapeDtypeStruct((n, D), jnp.float32),
    )(accsc, alphaL, sL, cmax_row, W_out, b_out)


def kernel(x, pos, edge_index, W_in, b_in, W_out, b_out, W_lin, W_src, W_dst,
           Wp1, bp1, Wp2, bp2, Wa1, ba1, Wa2, ba2):
    n, D = x.shape
    jj = edge_index[0].astype(jnp.int32)
    ii = edge_index[1].astype(jnp.int32)
    pos8 = jnp.pad(pos.astype(jnp.float32), ((0, 0), (0, 8 - pos.shape[1])))
    Wp1_8 = jnp.pad(Wp1, ((0, 8 - Wp1.shape[0]), (0, 0)))
    r1 = lambda v: v.reshape(1, -1)

    h, p1, sL, alphaL, cmaxA = _stage_a(
        n, x, pos8, W_in, r1(b_in), W_src, W_dst, W_lin, Wp1_8, r1(bp1), Wp2,
        r1(bp2), Wa1, r1(ba1), Wa2, r1(ba2))
    p1pad = jnp.pad(p1, ((0, 0), (0, D - p1.shape[1])))
    hi, hj, qi, qj = _gather_sc(h, p1pad, ii, jj)
    alpha, s_arr, cmaxC = _stage_c(
        hi, hj, qi, qj, W_src, W_dst, W_lin, r1(bp1), Wp2, r1(bp2), Wa1,
        r1(ba1), Wa2, r1(ba2))
    cmax = jnp.max(jnp.maximum(cmaxA, cmaxC), axis=0)
    npad = ((n + NS * 8 - 1) // (NS * 8)) * (NS * 8)
    zero = jnp.zeros((npad, D), jnp.float32)
    accsc = _scatter_sc(alpha, s_arr, ii, cmax.reshape(NC, D // NC), zero)
    return _stage_e(accsc, alphaL, sL, cmax.reshape(1, -1), W_out, r1(b_out))


# speedup vs baseline: 7.1464x; 1.4839x over previous
import functools

import jax
import jax.numpy as jnp
from jax import lax
from jax.experimental import pallas as pl
from jax.experimental.pallas import tpu as pltpu
from jax.experimental.pallas import tpu_sc as plsc

NC = 2
NS = 16
BN = 1000
KE = 2000
CK = 80
CKD = 40


def _stage_a_body(x_ref, pos_ref, W_in_ref, b_in_ref, W_src_ref, W_dst_ref,
                  W_lin_ref, Wp1_ref, bp1_ref, Wp2_ref, bp2_ref,
                  Wa1_ref, ba1_ref, Wa2_ref, ba2_ref,
                  t_ref, sL_ref, alphaL_ref, cmax_ref):
    pid = pl.program_id(0)
    D = x_ref.shape[1]
    h = jnp.maximum(x_ref[...] @ W_in_ref[...] + b_in_ref[...], 0.0)
    t_ref[:, 0:D] = h
    t_ref[:, D:2 * D] = pos_ref[...] @ Wp1_ref[...]
    d0 = jnp.maximum(jnp.maximum(bp1_ref[...], 0.0) @ Wp2_ref[...]
                     + bp2_ref[...], 0.0)
    sL_ref[...] = h @ W_lin_ref[...] + d0
    t = h @ W_dst_ref[...] - h @ W_src_ref[...] + d0
    a1 = jnp.maximum(t @ Wa1_ref[...] + ba1_ref[...], 0.0)
    aL = jnp.maximum(a1 @ Wa2_ref[...] + ba2_ref[...], 0.0)
    alphaL_ref[...] = aL

    @pl.when(pid == 0)
    def _():
        cmax_ref[...] = jnp.zeros_like(cmax_ref)

    cm = jnp.max(aL, axis=0, keepdims=True)
    cmax_ref[...] = jnp.maximum(cmax_ref[...], jnp.broadcast_to(cm, cmax_ref.shape))


def _stage_a(n, x, pos8, W_in, b_in, W_src, W_dst, W_lin, Wp1_p, bp1, Wp2, bp2,
             Wa1, ba1, Wa2, ba2):
    D = x.shape[1]
    H = bp1.shape[1]
    grid = (n // BN,)
    full = lambda shape: pl.BlockSpec(shape, lambda i: (0, 0))
    row = lambda w: pl.BlockSpec((BN, w), lambda i: (i, 0))
    return pl.pallas_call(
        _stage_a_body,
        grid=grid,
        in_specs=[row(D), pl.BlockSpec((BN, 8), lambda i: (i, 0)),
                  full((D, D)), full((1, D)), full((D, D)), full((D, D)),
                  full((D, D)), full((8, D)), full((1, H)), full((H, D)),
                  full((1, D)), full((D, H)), full((1, H)), full((H, D)),
                  full((1, D))],
        out_specs=[row(2 * D), row(D), row(D),
                   pl.BlockSpec((8, D), lambda i: (0, 0))],
        out_shape=[jax.ShapeDtypeStruct((n, 2 * D), jnp.float32),
                   jax.ShapeDtypeStruct((n, D), jnp.float32),
                   jax.ShapeDtypeStruct((n, D), jnp.float32),
                   jax.ShapeDtypeStruct((8, D), jnp.float32)],
    )(x, pos8, W_in, b_in, W_src, W_dst, W_lin, Wp1_p, bp1, Wp2, bp2,
      Wa1, ba1, Wa2, ba2)


def _gather_sc(tbl, ijt, E):
    n2, TW = tbl.shape
    D = TW // 2
    Hq = 64
    EW = E // (NC * NS)
    NCH = EW // CK

    @functools.partial(
        pl.kernel,
        out_type=[jax.ShapeDtypeStruct((E, D), jnp.float32),
                  jax.ShapeDtypeStruct((E, D), jnp.float32),
                  jax.ShapeDtypeStruct((E, Hq), jnp.float32)],
        mesh=plsc.VectorSubcoreMesh(core_axis_name="c", subcore_axis_name="s"),
        scratch_types=[pltpu.VMEM((2, CK), jnp.int32),
                       pltpu.VMEM((2, CK), jnp.int32),
                       pltpu.VMEM((CK, TW), jnp.float32),
                       pltpu.VMEM((CK, TW), jnp.float32),
                       pltpu.VMEM((CK, TW), jnp.float32),
                       pltpu.VMEM((CK, TW), jnp.float32),
                       pltpu.VMEM((CK, Hq), jnp.float32),
                       pltpu.VMEM((CK, Hq), jnp.float32),
                       pltpu.SemaphoreType.DMA,
                       pltpu.SemaphoreType.DMA,
                       pltpu.SemaphoreType.DMA,
                       pltpu.SemaphoreType.DMA],
    )
    def k(tbl_hbm, ijt_hbm, hi_hbm, hj_hbm, qd_hbm,
          ij0, ij1, bi0, bi1, bj0, bj1, qd0, qd1, gs0, gs1, ws0, ws1):
        ijv = (ij0, ij1)
        bi = (bi0, bi1)
        bj = (bj0, bj1)
        qdv = (qd0, qd1)
        gs = (gs0, gs1)
        ws = (ws0, ws1)
        wid = lax.axis_index("s") * NC + lax.axis_index("c")
        base = wid * EW
        bch = wid * NCH

        def fire(kk, b):
            pltpu.sync_copy(ijt_hbm.at[bch + kk], ijv[b])
            c1 = pltpu.async_copy(tbl_hbm.at[ijv[b].at[0]], bi[b], gs[b])
            c2 = pltpu.async_copy(tbl_hbm.at[ijv[b].at[1]], bj[b], gs[b])
            return c1, c2

        def emit(kk, b, c1, c2):
            c1.wait()
            c2.wait()

            def rowfn(r, cc):
                for rr in (2 * r, 2 * r + 1):
                    for g in range(Hq // 16):
                        o = pl.ds(D + g * 16, 16)
                        qdv[b][rr, pl.ds(g * 16, 16)] = bi[b][rr, o] - bj[b][rr, o]
                return cc

            lax.fori_loop(0, CK // 2, rowfn, 0)
            e0 = base + kk * CK
            pltpu.async_copy(bi[b].at[pl.ds(0, CK), pl.ds(0, D)],
                             hi_hbm.at[pl.ds(e0, CK)], ws[b])
            pltpu.async_copy(bj[b].at[pl.ds(0, CK), pl.ds(0, D)],
                             hj_hbm.at[pl.ds(e0, CK)], ws[b])
            pltpu.async_copy(qdv[b], qd_hbm.at[pl.ds(e0, CK)], ws[b])

        def drain_writes(b):
            pltpu.make_async_copy(bi[b].at[pl.ds(0, CK), pl.ds(0, D)],
                                  hi_hbm.at[pl.ds(0, CK)], ws[b]).wait()
            pltpu.make_async_copy(bj[b].at[pl.ds(0, CK), pl.ds(0, D)],
                                  hj_hbm.at[pl.ds(0, CK)], ws[b]).wait()
            pltpu.make_async_copy(qdv[b], qd_hbm.at[pl.ds(0, CK)], ws[b]).wait()

        c1, c2 = fire(0, 0)
        c3, c4 = fire(1, 1)
        emit(0, 0, c1, c2)
        emit(1, 1, c3, c4)

        def body(g, cc):
            drain_writes(0)
            a1, a2 = fire(2 * g, 0)
            drain_writes(1)
            b1, b2 = fire(2 * g + 1, 1)
            emit(2 * g, 0, a1, a2)
            emit(2 * g + 1, 1, b1, b2)
            return cc

        lax.fori_loop(1, NCH // 2, body, 0)
        drain_writes(0)
        e1, e2 = fire(NCH - 1, 0)
        emit(NCH - 1, 0, e1, e2)
        drain_writes(1)
        drain_writes(0)

    return k(tbl, ijt)


def _stage_c_body(hi_ref, hj_ref, qd_ref, W_src_ref, W_dst_ref,
                  W_lin_ref, bp1_ref, Wp2_ref, bp2_ref, Wa1_ref, ba1_ref,
                  Wa2_ref, ba2_ref, alpha_ref, s_ref, cmax_ref):
    pid = pl.program_id(0)
    hi = hi_ref[...]
    hj = hj_ref[...]
    g1 = qd_ref[...] + bp1_ref[...]
    delta = jnp.maximum(jnp.maximum(g1, 0.0) @ Wp2_ref[...] + bp2_ref[...], 0.0)
    t = hi @ W_dst_ref[...] - hj @ W_src_ref[...] + delta
    a1 = jnp.maximum(t @ Wa1_ref[...] + ba1_ref[...], 0.0)
    alpha = jnp.maximum(a1 @ Wa2_ref[...] + ba2_ref[...], 0.0)
    s = hj @ W_lin_ref[...] + delta
    Dh = alpha.shape[1] // 2
    alpha_ref[0] = alpha[:, 0:Dh]
    alpha_ref[1] = alpha[:, Dh:]
    s_ref[0] = s[:, 0:Dh]
    s_ref[1] = s[:, Dh:]

    @pl.when(pid == 0)
    def _():
        cmax_ref[...] = jnp.zeros_like(cmax_ref)

    cm = jnp.max(alpha, axis=0, keepdims=True)
    cmax_ref[...] = jnp.maximum(cmax_ref[...], jnp.broadcast_to(cm, cmax_ref.shape))


def _stage_c(hi, hj, qd, W_src, W_dst, W_lin, bp1, Wp2, bp2, Wa1, ba1,
             Wa2, ba2):
    E, D = hi.shape
    H = bp1.shape[1]
    grid = (E // KE,)
    full = lambda shape: pl.BlockSpec(shape, lambda i: (0, 0))
    row = lambda w: pl.BlockSpec((KE, w), lambda i: (i, 0))
    return pl.pallas_call(
        _stage_c_body,
        grid=grid,
        in_specs=[row(D), row(D), row(H),
                  full((D, D)), full((D, D)), full((D, D)), full((1, H)),
                  full((H, D)), full((1, D)), full((D, H)), full((1, H)),
                  full((H, D)), full((1, D))],
        out_specs=[pl.BlockSpec((NC, KE, D // 2), lambda i: (0, i, 0)),
                   pl.BlockSpec((NC, KE, D // 2), lambda i: (0, i, 0)),
                   pl.BlockSpec((8, D), lambda i: (0, 0))],
        out_shape=[jax.ShapeDtypeStruct((NC, E, D // 2), jnp.float32),
                   jax.ShapeDtypeStruct((NC, E, D // 2), jnp.float32),
                   jax.ShapeDtypeStruct((8, D), jnp.float32)],
    )(hi, hj, qd, W_src, W_dst, W_lin, bp1, Wp2, bp2, Wa1, ba1, Wa2, ba2)


def _scatter_sc(alpha, s_arr, ii2, cmax2, zero):
    _, E, Hh = alpha.shape
    D = 2 * Hh
    n = zero.shape[0]
    ESC = E // NS
    NCHD = ESC // CKD
    NPS = n // NS
    RPC = 1

    @functools.partial(
        pl.kernel,
        out_type=jax.ShapeDtypeStruct((NC, n, D), jnp.float32),
        mesh=plsc.VectorSubcoreMesh(core_axis_name="c", subcore_axis_name="s"),
        scratch_types=[pltpu.VMEM((RPC, CKD), jnp.int32),
                       pltpu.VMEM((RPC, CKD), jnp.int32),
                       pltpu.VMEM((CKD, Hh), jnp.float32),
                       pltpu.VMEM((CKD, Hh), jnp.float32),
                       pltpu.VMEM((CKD, Hh), jnp.float32),
                       pltpu.VMEM((CKD, Hh), jnp.float32),
                       pltpu.VMEM((CKD, D), jnp.float32),
                       pltpu.VMEM((Hh,), jnp.float32),
                       pltpu.VMEM_SHARED((n, D), jnp.float32),
                       pltpu.SemaphoreType.DMA,
                       pltpu.SemaphoreType.DMA],
    )
    def k(alpha_hbm, s_hbm, ii2_hbm, cmax_hbm, zero_hbm, out_hbm,
          ix0, ix1, av0, av1, sv0, sv1, py0, cm_v, acc_sh, rs0, rs1):
        ixv = (ix0, ix1)
        av = (av0, av1)
        sv = (sv0, sv1)
        pay = (py0, py0)
        rs = (rs0, rs1)
        c = lax.axis_index("c")
        sid = lax.axis_index("s")
        pltpu.sync_copy(zero_hbm.at[pl.ds(sid * NPS, NPS)],
                        acc_sh.at[pl.ds(sid * NPS, NPS)])
        pltpu.sync_copy(cmax_hbm.at[c], cm_v)
        plsc.subcore_barrier()
        cms = [cm_v[pl.ds(g * 16, 16)] for g in range(Hh // 16)]

        def fire(kk, b):
            e0 = sid * ESC + kk * CKD
            r0 = sid * NCHD + kk
            pltpu.async_copy(ii2_hbm.at[pl.ds(r0, RPC)], ixv[b], rs[b])
            pltpu.async_copy(alpha_hbm.at[c, pl.ds(e0, CKD)], av[b], rs[b])
            pltpu.async_copy(s_hbm.at[c, pl.ds(e0, CKD)], sv[b], rs[b])

        def drain_reads(b):
            pltpu.make_async_copy(ii2_hbm.at[pl.ds(0, RPC)], ixv[b], rs[b]).wait()
            pltpu.make_async_copy(alpha_hbm.at[c, pl.ds(0, CKD)], av[b],
                                  rs[b]).wait()
            pltpu.make_async_copy(s_hbm.at[c, pl.ds(0, CKD)], sv[b],
                                  rs[b]).wait()

        def compute_and_scatter(b):
            def rowfn(r, cc):
                for rr in (2 * r, 2 * r + 1):
                    for g in range(Hh // 16):
                        ex = jnp.exp(av[b][rr, pl.ds(g * 16, 16)] - cms[g])
                        pay[b][rr, pl.ds(Hh + g * 16, 16)] = ex
                        pay[b][rr, pl.ds(g * 16, 16)] = \
                            ex * sv[b][rr, pl.ds(g * 16, 16)]
                return cc

            lax.fori_loop(0, CKD // 2, rowfn, 0)
            pltpu.sync_copy(pay[b], acc_sh.at[ixv[b].at[0]], add=True)

        fire(0, 0)
        fire(1, 1)

        def body(g, cc):
            drain_reads(0)
            compute_and_scatter(0)
            nxt0 = 2 * g + 2

            @pl.when(nxt0 < NCHD)
            def _():
                fire(nxt0, 0)

            drain_reads(1)
            compute_and_scatter(1)
            nxt1 = 2 * g + 3

            @pl.when(nxt1 < NCHD)
            def _():
                fire(nxt1, 1)

            return cc

        lax.fori_loop(0, NCHD // 2, body, 0)
        if NCHD % 2 == 1:
            drain_reads(0)
            compute_and_scatter(0)
        plsc.subcore_barrier()
        pltpu.sync_copy(acc_sh.at[pl.ds(sid * NPS, NPS)],
                        out_hbm.at[c, pl.ds(sid * NPS, NPS)])

    return k(alpha, s_arr, ii2, cmax2, zero)


def _stage_e_body(acc_ref, alphaL_ref, sL_ref, cmax_ref, W_out_ref, b_out_ref,
                  o_ref):
    D = o_ref.shape[1]
    Hh = D // 2
    exL = jnp.exp(alphaL_ref[...] - cmax_ref[...])
    sL = sL_ref[...]
    num0 = acc_ref[0, :, 0:Hh] + exL[:, 0:Hh] * sL[:, 0:Hh]
    den0 = acc_ref[0, :, Hh:D] + exL[:, 0:Hh]
    num1 = acc_ref[1, :, 0:Hh] + exL[:, Hh:D] * sL[:, Hh:D]
    den1 = acc_ref[1, :, Hh:D] + exL[:, Hh:D]
    o0 = num0 / (den0 + 1e-16)
    o1 = num1 / (den1 + 1e-16)
    out = (o0 @ W_out_ref[0:Hh, :] + o1 @ W_out_ref[Hh:D, :]) + b_out_ref[...]
    o_ref[...] = jnp.maximum(out, 0.0)


def _stage_e(accsc, alphaL, sL, cmax_row, W_out, b_out):
    n, D = alphaL.shape
    grid = (n // BN,)
    full = lambda shape: pl.BlockSpec(shape, lambda i: (0, 0))
    row = lambda w: pl.BlockSpec((BN, w), lambda i: (i, 0))
    return pl.pallas_call(
        _stage_e_body,
        grid=grid,
        in_specs=[pl.BlockSpec((NC, BN, D), lambda i: (0, i, 0)),
                  row(D), row(D), full((1, D)), full((D, D)), full((1, D))],
        out_specs=row(D),
        out_shape=jax.ShapeDtypeStruct((n, D), jnp.float32),
    )(accsc, alphaL, sL, cmax_row, W_out, b_out)


def kernel(x, pos, edge_index, W_in, b_in, W_out, b_out, W_lin, W_src, W_dst,
           Wp1, bp1, Wp2, bp2, Wa1, ba1, Wa2, ba2):
    n, D = x.shape
    E = edge_index.shape[1]
    jj = edge_index[0].astype(jnp.int32)
    ii = edge_index[1].astype(jnp.int32)
    pos8 = jnp.pad(pos.astype(jnp.float32), ((0, 0), (0, 8 - pos.shape[1])))
    Wp1_p = jnp.pad(Wp1, ((0, 8 - Wp1.shape[0]), (0, D - Wp1.shape[1])))
    r1 = lambda v: v.reshape(1, -1)

    tbl, sL, alphaL, cmaxA = _stage_a(
        n, x, pos8, W_in, r1(b_in), W_src, W_dst, W_lin, Wp1_p, r1(bp1), Wp2,
        r1(bp2), Wa1, r1(ba1), Wa2, r1(ba2))
    ijt = jnp.stack([ii.reshape(E // CK, CK), jj.reshape(E // CK, CK)], axis=1)
    hi, hj, qd = _gather_sc(tbl, ijt, E)
    alpha, s_arr, cmaxC = _stage_c(
        hi, hj, qd, W_src, W_dst, W_lin, r1(bp1), Wp2, r1(bp2), Wa1,
        r1(ba1), Wa2, r1(ba2))
    cmax = jnp.max(jnp.maximum(cmaxA, cmaxC), axis=0)
    npad = ((n + NS * 8 - 1) // (NS * 8)) * (NS * 8)
    zero = jnp.zeros((npad, D), jnp.float32)
    accsc = _scatter_sc(alpha, s_arr, ii.reshape(E // CKD, CKD),
                        cmax.reshape(NC, D // NC), zero)
    return _stage_e(accsc, alphaL, sL, cmax.reshape(1, -1), W_out, r1(b_out))


# speedup vs baseline: 7.6724x; 1.0736x over previous
import functools

import jax
import jax.numpy as jnp
from jax import lax
from jax.experimental import pallas as pl
from jax.experimental.pallas import tpu as pltpu
from jax.experimental.pallas import tpu_sc as plsc

NC = 2
NS = 16
BN = 1000
KE = 1280
CK = 80
CKD = 40


def _stage_a_body(x_ref, pos_ref, W_in_ref, b_in_ref, W_src_ref, W_dst_ref,
                  W_lin_ref, Wp1_ref, bp1_ref, Wp2_ref, bp2_ref,
                  Wa1_ref, ba1_ref, Wa2_ref, ba2_ref,
                  t_ref, sL_ref, alphaL_ref, cmax_ref):
    pid = pl.program_id(0)
    D = x_ref.shape[1]
    h = jnp.maximum(x_ref[...] @ W_in_ref[...] + b_in_ref[...], 0.0)
    t_ref[:, 0:D] = h
    t_ref[:, D:2 * D] = pos_ref[...] @ Wp1_ref[...]
    d0 = jnp.maximum(jnp.maximum(bp1_ref[...], 0.0) @ Wp2_ref[...]
                     + bp2_ref[...], 0.0)
    sL_ref[...] = h @ W_lin_ref[...] + d0
    t = h @ W_dst_ref[...] - h @ W_src_ref[...] + d0
    a1 = jnp.maximum(t @ Wa1_ref[...] + ba1_ref[...], 0.0)
    aL = jnp.maximum(a1 @ Wa2_ref[...] + ba2_ref[...], 0.0)
    alphaL_ref[...] = aL

    @pl.when(pid == 0)
    def _():
        cmax_ref[...] = jnp.zeros_like(cmax_ref)

    cm = jnp.max(aL, axis=0, keepdims=True)
    cmax_ref[...] = jnp.maximum(cmax_ref[...], jnp.broadcast_to(cm, cmax_ref.shape))


def _stage_a(n, x, pos8, W_in, b_in, W_src, W_dst, W_lin, Wp1_p, bp1, Wp2, bp2,
             Wa1, ba1, Wa2, ba2):
    D = x.shape[1]
    H = bp1.shape[1]
    grid = (n // BN,)
    full = lambda shape: pl.BlockSpec(shape, lambda i: (0, 0))
    row = lambda w: pl.BlockSpec((BN, w), lambda i: (i, 0))
    return pl.pallas_call(
        _stage_a_body,
        grid=grid,
        in_specs=[row(D), pl.BlockSpec((BN, 8), lambda i: (i, 0)),
                  full((D, D)), full((1, D)), full((D, D)), full((D, D)),
                  full((D, D)), full((8, D)), full((1, H)), full((H, D)),
                  full((1, D)), full((D, H)), full((1, H)), full((H, D)),
                  full((1, D))],
        out_specs=[row(2 * D), row(D), row(D),
                   pl.BlockSpec((8, D), lambda i: (0, 0))],
        out_shape=[jax.ShapeDtypeStruct((n, 2 * D), jnp.float32),
                   jax.ShapeDtypeStruct((n, D), jnp.float32),
                   jax.ShapeDtypeStruct((n, D), jnp.float32),
                   jax.ShapeDtypeStruct((8, D), jnp.float32)],
    )(x, pos8, W_in, b_in, W_src, W_dst, W_lin, Wp1_p, bp1, Wp2, bp2,
      Wa1, ba1, Wa2, ba2)


def _gather_sc(tbl, ijt, E):
    n2, TW = tbl.shape
    D = TW // 2
    Hq = 64
    EW = E // (NC * NS)
    NCH = EW // CK

    @functools.partial(
        pl.kernel,
        out_type=[jax.ShapeDtypeStruct((E, D), jnp.float32),
                  jax.ShapeDtypeStruct((E, D), jnp.float32),
                  jax.ShapeDtypeStruct((E, Hq), jnp.float32)],
        mesh=plsc.VectorSubcoreMesh(core_axis_name="c", subcore_axis_name="s"),
        scratch_types=[pltpu.VMEM((2, CK), jnp.int32),
                       pltpu.VMEM((2, CK), jnp.int32),
                       pltpu.VMEM((CK, TW), jnp.float32),
                       pltpu.VMEM((CK, TW), jnp.float32),
                       pltpu.VMEM((CK, TW), jnp.float32),
                       pltpu.VMEM((CK, TW), jnp.float32),
                       pltpu.VMEM((CK, Hq), jnp.float32),
                       pltpu.VMEM((CK, Hq), jnp.float32),
                       pltpu.SemaphoreType.DMA,
                       pltpu.SemaphoreType.DMA,
                       pltpu.SemaphoreType.DMA,
                       pltpu.SemaphoreType.DMA],
    )
    def k(tbl_hbm, ijt_hbm, hi_hbm, hj_hbm, qd_hbm,
          ij0, ij1, bi0, bi1, bj0, bj1, qd0, qd1, gs0, gs1, ws0, ws1):
        ijv = (ij0, ij1)
        bi = (bi0, bi1)
        bj = (bj0, bj1)
        qdv = (qd0, qd1)
        gs = (gs0, gs1)
        ws = (ws0, ws1)
        wid = lax.axis_index("s") * NC + lax.axis_index("c")
        base = wid * EW
        bch = wid * NCH

        def fire(kk, b):
            pltpu.sync_copy(ijt_hbm.at[bch + kk], ijv[b])
            c1 = pltpu.async_copy(tbl_hbm.at[ijv[b].at[0]], bi[b], gs[b])
            c2 = pltpu.async_copy(tbl_hbm.at[ijv[b].at[1]], bj[b], gs[b])
            return c1, c2

        def emit(kk, b, c1, c2):
            c1.wait()
            c2.wait()

            def rowfn(r, cc):
                for rr in (2 * r, 2 * r + 1):
                    for g in range(Hq // 16):
                        o = pl.ds(D + g * 16, 16)
                        qdv[b][rr, pl.ds(g * 16, 16)] = bi[b][rr, o] - bj[b][rr, o]
                return cc

            lax.fori_loop(0, CK // 2, rowfn, 0)
            e0 = base + kk * CK
            pltpu.async_copy(bi[b].at[pl.ds(0, CK), pl.ds(0, D)],
                             hi_hbm.at[pl.ds(e0, CK)], ws[b])
            pltpu.async_copy(bj[b].at[pl.ds(0, CK), pl.ds(0, D)],
                             hj_hbm.at[pl.ds(e0, CK)], ws[b])
            pltpu.async_copy(qdv[b], qd_hbm.at[pl.ds(e0, CK)], ws[b])

        def drain_writes(b):
            pltpu.make_async_copy(bi[b].at[pl.ds(0, CK), pl.ds(0, D)],
                                  hi_hbm.at[pl.ds(0, CK)], ws[b]).wait()
            pltpu.make_async_copy(bj[b].at[pl.ds(0, CK), pl.ds(0, D)],
                                  hj_hbm.at[pl.ds(0, CK)], ws[b]).wait()
            pltpu.make_async_copy(qdv[b], qd_hbm.at[pl.ds(0, CK)], ws[b]).wait()

        c1, c2 = fire(0, 0)
        c3, c4 = fire(1, 1)
        emit(0, 0, c1, c2)
        emit(1, 1, c3, c4)

        def body(g, cc):
            drain_writes(0)
            a1, a2 = fire(2 * g, 0)
            drain_writes(1)
            b1, b2 = fire(2 * g + 1, 1)
            emit(2 * g, 0, a1, a2)
            emit(2 * g + 1, 1, b1, b2)
            return cc

        lax.fori_loop(1, NCH // 2, body, 0)
        if NCH % 2 == 1:
            drain_writes(0)
            e1, e2 = fire(NCH - 1, 0)
            emit(NCH - 1, 0, e1, e2)
        drain_writes(1)
        drain_writes(0)

    return k(tbl, ijt)


def _stage_c_body(hi_ref, hj_ref, qd_ref, W_src_ref, W_dst_ref,
                  W_lin_ref, bp1_ref, Wp2_ref, bp2_ref, Wa1_ref, ba1_ref,
                  Wa2_ref, ba2_ref, alpha_ref, s_ref):
    hi = hi_ref[...]
    hj = hj_ref[...]
    g1 = qd_ref[...] + bp1_ref[...]
    delta = jnp.maximum(jnp.maximum(g1, 0.0) @ Wp2_ref[...] + bp2_ref[...], 0.0)
    t = hi @ W_dst_ref[...] - hj @ W_src_ref[...] + delta
    a1 = jnp.maximum(t @ Wa1_ref[...] + ba1_ref[...], 0.0)
    alpha = jnp.maximum(a1 @ Wa2_ref[...] + ba2_ref[...], 0.0)
    s = hj @ W_lin_ref[...] + delta
    Dh = alpha.shape[1] // 2
    alpha_ref[0] = alpha[:, 0:Dh]
    alpha_ref[1] = alpha[:, Dh:]
    s_ref[0] = s[:, 0:Dh]
    s_ref[1] = s[:, Dh:]


def _stage_c(hi, hj, qd, W_src, W_dst, W_lin, bp1, Wp2, bp2, Wa1, ba1,
             Wa2, ba2):
    E, D = hi.shape
    H = bp1.shape[1]
    grid = (E // KE,)
    full = lambda shape: pl.BlockSpec(shape, lambda i: (0, 0))
    row = lambda w: pl.BlockSpec((KE, w), lambda i: (i, 0))
    return pl.pallas_call(
        _stage_c_body,
        grid=grid,
        in_specs=[row(D), row(D), row(H),
                  full((D, D)), full((D, D)), full((D, D)), full((1, H)),
                  full((H, D)), full((1, D)), full((D, H)), full((1, H)),
                  full((H, D)), full((1, D))],
        out_specs=[pl.BlockSpec((NC, KE, D // 2), lambda i: (0, i, 0)),
                   pl.BlockSpec((NC, KE, D // 2), lambda i: (0, i, 0))],
        out_shape=[jax.ShapeDtypeStruct((NC, E, D // 2), jnp.float32),
                   jax.ShapeDtypeStruct((NC, E, D // 2), jnp.float32)],
    )(hi, hj, qd, W_src, W_dst, W_lin, bp1, Wp2, bp2, Wa1, ba1, Wa2, ba2)


def _scatter_sc(alpha, s_arr, ii2, cmax2, init):
    _, E, Hh = alpha.shape
    D = 2 * Hh
    n = init.shape[1]
    ESC = E // NS
    NCHD = ESC // CKD
    NPS = n // NS
    RPC = 1

    @functools.partial(
        pl.kernel,
        out_type=jax.ShapeDtypeStruct((NC, n, D), jnp.float32),
        mesh=plsc.VectorSubcoreMesh(core_axis_name="c", subcore_axis_name="s"),
        scratch_types=[pltpu.VMEM((RPC, CKD), jnp.int32),
                       pltpu.VMEM((RPC, CKD), jnp.int32),
                       pltpu.VMEM((CKD, Hh), jnp.float32),
                       pltpu.VMEM((CKD, Hh), jnp.float32),
                       pltpu.VMEM((CKD, Hh), jnp.float32),
                       pltpu.VMEM((CKD, Hh), jnp.float32),
                       pltpu.VMEM((CKD, D), jnp.float32),
                       pltpu.VMEM((Hh,), jnp.float32),
                       pltpu.VMEM_SHARED((n, D), jnp.float32),
                       pltpu.SemaphoreType.DMA,
                       pltpu.SemaphoreType.DMA],
    )
    def k(alpha_hbm, s_hbm, ii2_hbm, cmax_hbm, init_hbm, out_hbm,
          ix0, ix1, av0, av1, sv0, sv1, py0, cm_v, acc_sh, rs0, rs1):
        ixv = (ix0, ix1)
        av = (av0, av1)
        sv = (sv0, sv1)
        pay = (py0, py0)
        rs = (rs0, rs1)
        c = lax.axis_index("c")
        sid = lax.axis_index("s")
        pltpu.sync_copy(init_hbm.at[c, pl.ds(sid * NPS, NPS)],
                        acc_sh.at[pl.ds(sid * NPS, NPS)])
        pltpu.sync_copy(cmax_hbm.at[c], cm_v)
        plsc.subcore_barrier()
        cms = [cm_v[pl.ds(g * 16, 16)] for g in range(Hh // 16)]

        def fire(kk, b):
            e0 = sid * ESC + kk * CKD
            r0 = sid * NCHD + kk
            pltpu.async_copy(ii2_hbm.at[pl.ds(r0, RPC)], ixv[b], rs[b])
            pltpu.async_copy(alpha_hbm.at[c, pl.ds(e0, CKD)], av[b], rs[b])
            pltpu.async_copy(s_hbm.at[c, pl.ds(e0, CKD)], sv[b], rs[b])

        def drain_reads(b):
            pltpu.make_async_copy(ii2_hbm.at[pl.ds(0, RPC)], ixv[b], rs[b]).wait()
            pltpu.make_async_copy(alpha_hbm.at[c, pl.ds(0, CKD)], av[b],
                                  rs[b]).wait()
            pltpu.make_async_copy(s_hbm.at[c, pl.ds(0, CKD)], sv[b],
                                  rs[b]).wait()

        def compute_and_scatter(b):
            def rowfn(r, cc):
                for rr in (2 * r, 2 * r + 1):
                    for g in range(Hh // 16):
                        ex = jnp.exp(av[b][rr, pl.ds(g * 16, 16)] - cms[g])
                        pay[b][rr, pl.ds(Hh + g * 16, 16)] = ex
                        pay[b][rr, pl.ds(g * 16, 16)] = \
                            ex * sv[b][rr, pl.ds(g * 16, 16)]
                return cc

            lax.fori_loop(0, CKD // 2, rowfn, 0)
            pltpu.sync_copy(pay[b], acc_sh.at[ixv[b].at[0]], add=True)

        fire(0, 0)
        fire(1, 1)

        def body(g, cc):
            drain_reads(0)
            compute_and_scatter(0)
            nxt0 = 2 * g + 2

            @pl.when(nxt0 < NCHD)
            def _():
                fire(nxt0, 0)

            drain_reads(1)
            compute_and_scatter(1)
            nxt1 = 2 * g + 3

            @pl.when(nxt1 < NCHD)
            def _():
                fire(nxt1, 1)

            return cc

        lax.fori_loop(0, NCHD // 2, body, 0)
        if NCHD % 2 == 1:
            drain_reads(0)
            compute_and_scatter(0)
        plsc.subcore_barrier()
        pltpu.sync_copy(acc_sh.at[pl.ds(sid * NPS, NPS)],
                        out_hbm.at[c, pl.ds(sid * NPS, NPS)])

    return k(alpha, s_arr, ii2, cmax2, init)


def _stage_e_body(acc_ref, alphaL_ref, sL_ref, cmax_ref, W_out_ref, b_out_ref,
                  o_ref):
    D = o_ref.shape[1]
    Hh = D // 2
    exL = jnp.exp(alphaL_ref[...] - cmax_ref[...])
    sL = sL_ref[...]
    num0 = acc_ref[0, :, 0:Hh] + exL[:, 0:Hh] * sL[:, 0:Hh]
    den0 = acc_ref[0, :, Hh:D] + exL[:, 0:Hh]
    num1 = acc_ref[1, :, 0:Hh] + exL[:, Hh:D] * sL[:, Hh:D]
    den1 = acc_ref[1, :, Hh:D] + exL[:, Hh:D]
    o0 = num0 / (den0 + 1e-16)
    o1 = num1 / (den1 + 1e-16)
    out = (o0 @ W_out_ref[0:Hh, :] + o1 @ W_out_ref[Hh:D, :]) + b_out_ref[...]
    o_ref[...] = jnp.maximum(out, 0.0)


def _stage_e(accsc, alphaL, sL, cmax_row, W_out, b_out):
    n, D = alphaL.shape
    grid = (n // BN,)
    full = lambda shape: pl.BlockSpec(shape, lambda i: (0, 0))
    row = lambda w: pl.BlockSpec((BN, w), lambda i: (i, 0))
    return pl.pallas_call(
        _stage_e_body,
        grid=grid,
        in_specs=[pl.BlockSpec((NC, BN, D), lambda i: (0, i, 0)),
                  row(D), row(D), full((1, D)), full((D, D)), full((1, D))],
        out_specs=row(D),
        out_shape=jax.ShapeDtypeStruct((n, D), jnp.float32),
    )(accsc, alphaL, sL, cmax_row, W_out, b_out)


def kernel(x, pos, edge_index, W_in, b_in, W_out, b_out, W_lin, W_src, W_dst,
           Wp1, bp1, Wp2, bp2, Wa1, ba1, Wa2, ba2):
    n, D = x.shape
    E = edge_index.shape[1]
    jj = edge_index[0].astype(jnp.int32)
    ii = edge_index[1].astype(jnp.int32)
    pos8 = jnp.pad(pos.astype(jnp.float32), ((0, 0), (0, 8 - pos.shape[1])))
    Wp1_p = jnp.pad(Wp1, ((0, 8 - Wp1.shape[0]), (0, D - Wp1.shape[1])))
    r1 = lambda v: v.reshape(1, -1)

    tbl, sL, alphaL, cmaxA = _stage_a(
        n, x, pos8, W_in, r1(b_in), W_src, W_dst, W_lin, Wp1_p, r1(bp1), Wp2,
        r1(bp2), Wa1, r1(ba1), Wa2, r1(ba2))
    cmax = jnp.max(cmaxA, axis=0)
    npad = ((n + NS * 8 - 1) // (NS * 8)) * (NS * 8)
    accsc = jnp.zeros((NC, npad, D), jnp.float32)

    unit = 32 * CK
    nu = E // unit
    sl = [(nu // 4 + (1 if t < nu % 4 else 0)) * unit for t in range(4)]
    a0 = 0
    for Es in sl:
        iis = lax.dynamic_slice_in_dim(ii, a0, Es)
        jjs = lax.dynamic_slice_in_dim(jj, a0, Es)
        a0 += Es
        ijt = jnp.stack([iis.reshape(Es // CK, CK),
                         jjs.reshape(Es // CK, CK)], axis=1)
        hi, hj, qd = _gather_sc(tbl, ijt, Es)
        alpha, s_arr = _stage_c(
            hi, hj, qd, W_src, W_dst, W_lin, r1(bp1), Wp2, r1(bp2), Wa1,
            r1(ba1), Wa2, r1(ba2))
        accsc = _scatter_sc(alpha, s_arr, iis.reshape(Es // CKD, CKD),
                            cmax.reshape(NC, D // NC), accsc)
    return _stage_e(accsc, alphaL, sL, cmax.reshape(1, -1), W_out, r1(b_out))


# speedup vs baseline: 8.9354x; 1.1646x over previous
import functools

import jax
import jax.numpy as jnp
from jax import lax
from jax.experimental import pallas as pl
from jax.experimental.pallas import tpu as pltpu
from jax.experimental.pallas import tpu_sc as plsc

NC = 2
NS = 16
BN = 1000
KE = 1280
CK = 80
CKD = 40


def _stage_a_body(x_ref, pos_ref, W_in_ref, b_in_ref, W_src_ref, W_dst_ref,
                  W_lin_ref, Wp1_ref, bp1_ref, Wp2_ref, bp2_ref,
                  Wa1_ref, ba1_ref, Wa2_ref, ba2_ref,
                  t_ref, sL_ref, alphaL_ref, cmax_ref):
    pid = pl.program_id(0)
    D = x_ref.shape[1]
    h = jnp.maximum(x_ref[...] @ W_in_ref[...] + b_in_ref[...], 0.0)
    t_ref[:, 0:D] = h
    t_ref[:, D:2 * D] = pos_ref[...] @ Wp1_ref[...]
    d0 = jnp.maximum(jnp.maximum(bp1_ref[...], 0.0) @ Wp2_ref[...]
                     + bp2_ref[...], 0.0)
    sL_ref[...] = h @ W_lin_ref[...] + d0
    t = h @ W_dst_ref[...] - h @ W_src_ref[...] + d0
    a1 = jnp.maximum(t @ Wa1_ref[...] + ba1_ref[...], 0.0)
    aL = jnp.maximum(a1 @ Wa2_ref[...] + ba2_ref[...], 0.0)
    alphaL_ref[...] = aL

    @pl.when(pid == 0)
    def _():
        cmax_ref[...] = jnp.zeros_like(cmax_ref)

    cm = jnp.max(aL, axis=0, keepdims=True)
    cmax_ref[...] = jnp.maximum(cmax_ref[...], jnp.broadcast_to(cm, cmax_ref.shape))


def _stage_a(n, x, pos8, W_in, b_in, W_src, W_dst, W_lin, Wp1_p, bp1, Wp2, bp2,
             Wa1, ba1, Wa2, ba2):
    D = x.shape[1]
    H = bp1.shape[1]
    grid = (n // BN,)
    full = lambda shape: pl.BlockSpec(shape, lambda i: (0, 0))
    row = lambda w: pl.BlockSpec((BN, w), lambda i: (i, 0))
    return pl.pallas_call(
        _stage_a_body,
        grid=grid,
        in_specs=[row(D), pl.BlockSpec((BN, 8), lambda i: (i, 0)),
                  full((D, D)), full((1, D)), full((D, D)), full((D, D)),
                  full((D, D)), full((8, D)), full((1, H)), full((H, D)),
                  full((1, D)), full((D, H)), full((1, H)), full((H, D)),
                  full((1, D))],
        out_specs=[row(2 * D), row(D), row(D),
                   pl.BlockSpec((8, D), lambda i: (0, 0))],
        out_shape=[jax.ShapeDtypeStruct((n, 2 * D), jnp.float32),
                   jax.ShapeDtypeStruct((n, D), jnp.float32),
                   jax.ShapeDtypeStruct((n, D), jnp.float32),
                   jax.ShapeDtypeStruct((8, D), jnp.float32)],
    )(x, pos8, W_in, b_in, W_src, W_dst, W_lin, Wp1_p, bp1, Wp2, bp2,
      Wa1, ba1, Wa2, ba2)


def _gather_sc(tbl, ijt, E):
    n2, TW = tbl.shape
    D = TW // 2
    Hq = 64
    EW = E // (NC * NS)
    NCH = EW // CK

    @functools.partial(
        pl.kernel,
        out_type=[jax.ShapeDtypeStruct((E, D), jnp.float32),
                  jax.ShapeDtypeStruct((E, D), jnp.float32),
                  jax.ShapeDtypeStruct((E, Hq), jnp.float32)],
        mesh=plsc.VectorSubcoreMesh(core_axis_name="c", subcore_axis_name="s"),
        scratch_types=[pltpu.VMEM((2, CK), jnp.int32),
                       pltpu.VMEM((2, CK), jnp.int32),
                       pltpu.VMEM((CK, TW), jnp.float32),
                       pltpu.VMEM((CK, TW), jnp.float32),
                       pltpu.VMEM((CK, TW), jnp.float32),
                       pltpu.VMEM((CK, TW), jnp.float32),
                       pltpu.VMEM((CK, Hq), jnp.float32),
                       pltpu.VMEM((CK, Hq), jnp.float32),
                       pltpu.SemaphoreType.DMA,
                       pltpu.SemaphoreType.DMA,
                       pltpu.SemaphoreType.DMA,
                       pltpu.SemaphoreType.DMA],
    )
    def k(tbl_hbm, ijt_hbm, hi_hbm, hj_hbm, qd_hbm,
          ij0, ij1, bi0, bi1, bj0, bj1, qd0, qd1, gs0, gs1, ws0, ws1):
        ijv = (ij0, ij1)
        bi = (bi0, bi1)
        bj = (bj0, bj1)
        qdv = (qd0, qd1)
        gs = (gs0, gs1)
        ws = (ws0, ws1)
        wid = lax.axis_index("s") * NC + lax.axis_index("c")
        base = wid * EW
        bch = wid * NCH

        def fire(kk, b):
            pltpu.sync_copy(ijt_hbm.at[bch + kk], ijv[b])
            c1 = pltpu.async_copy(tbl_hbm.at[ijv[b].at[0]], bi[b], gs[b])
            c2 = pltpu.async_copy(tbl_hbm.at[ijv[b].at[1]], bj[b], gs[b])
            return c1, c2

        def emit(kk, b, c1, c2):
            c1.wait()
            c2.wait()

            def rowfn(r, cc):
                for rr in (2 * r, 2 * r + 1):
                    for g in range(Hq // 16):
                        o = pl.ds(D + g * 16, 16)
                        qdv[b][rr, pl.ds(g * 16, 16)] = bi[b][rr, o] - bj[b][rr, o]
                return cc

            lax.fori_loop(0, CK // 2, rowfn, 0)
            e0 = base + kk * CK
            pltpu.async_copy(bi[b].at[pl.ds(0, CK), pl.ds(0, D)],
                             hi_hbm.at[pl.ds(e0, CK)], ws[b])
            pltpu.async_copy(bj[b].at[pl.ds(0, CK), pl.ds(0, D)],
                             hj_hbm.at[pl.ds(e0, CK)], ws[b])
            pltpu.async_copy(qdv[b], qd_hbm.at[pl.ds(e0, CK)], ws[b])

        def drain_writes(b):
            pltpu.make_async_copy(bi[b].at[pl.ds(0, CK), pl.ds(0, D)],
                                  hi_hbm.at[pl.ds(0, CK)], ws[b]).wait()
            pltpu.make_async_copy(bj[b].at[pl.ds(0, CK), pl.ds(0, D)],
                                  hj_hbm.at[pl.ds(0, CK)], ws[b]).wait()
            pltpu.make_async_copy(qdv[b], qd_hbm.at[pl.ds(0, CK)], ws[b]).wait()

        c1, c2 = fire(0, 0)
        c3, c4 = fire(1, 1)
        emit(0, 0, c1, c2)
        emit(1, 1, c3, c4)

        def body(g, cc):
            drain_writes(0)
            a1, a2 = fire(2 * g, 0)
            drain_writes(1)
            b1, b2 = fire(2 * g + 1, 1)
            emit(2 * g, 0, a1, a2)
            emit(2 * g + 1, 1, b1, b2)
            return cc

        lax.fori_loop(1, NCH // 2, body, 0)
        if NCH % 2 == 1:
            drain_writes(0)
            e1, e2 = fire(NCH - 1, 0)
            emit(NCH - 1, 0, e1, e2)
        drain_writes(1)
        drain_writes(0)

    return k(tbl, ijt)


def _stage_c_body(hi_ref, hj_ref, qd_ref, W_src_ref, W_dst_ref,
                  W_lin_ref, bp1_ref, Wp2_ref, bp2_ref, Wa1_ref, ba1_ref,
                  Wa2_ref, ba2_ref, comb_ref):
    hi = hi_ref[...]
    hj = hj_ref[...]
    g1 = qd_ref[...] + bp1_ref[...]
    delta = jnp.maximum(jnp.maximum(g1, 0.0) @ Wp2_ref[...] + bp2_ref[...], 0.0)
    t = hi @ W_dst_ref[...] - hj @ W_src_ref[...] + delta
    a1 = jnp.maximum(t @ Wa1_ref[...] + ba1_ref[...], 0.0)
    alpha = jnp.maximum(a1 @ Wa2_ref[...] + ba2_ref[...], 0.0)
    s = hj @ W_lin_ref[...] + delta
    Dh = alpha.shape[1] // 2
    comb_ref[0] = jnp.concatenate([alpha[:, 0:Dh], s[:, 0:Dh]], axis=1)
    comb_ref[1] = jnp.concatenate([alpha[:, Dh:], s[:, Dh:]], axis=1)


def _stage_c(hi, hj, qd, W_src, W_dst, W_lin, bp1, Wp2, bp2, Wa1, ba1,
             Wa2, ba2):
    E, D = hi.shape
    H = bp1.shape[1]
    grid = (E // KE,)
    full = lambda shape: pl.BlockSpec(shape, lambda i: (0, 0))
    row = lambda w: pl.BlockSpec((KE, w), lambda i: (i, 0))
    return pl.pallas_call(
        _stage_c_body,
        grid=grid,
        in_specs=[row(D), row(D), row(H),
                  full((D, D)), full((D, D)), full((D, D)), full((1, H)),
                  full((H, D)), full((1, D)), full((D, H)), full((1, H)),
                  full((H, D)), full((1, D))],
        out_specs=pl.BlockSpec((NC, KE, D), lambda i: (0, i, 0)),
        out_shape=jax.ShapeDtypeStruct((NC, E, D), jnp.float32),
    )(hi, hj, qd, W_src, W_dst, W_lin, bp1, Wp2, bp2, Wa1, ba1, Wa2, ba2)


def _scatter_sc(comb, ii2, cmax2, init):
    _, E, D = comb.shape
    Hh = D // 2
    n = init.shape[1]
    ESC = E // NS
    NCHD = ESC // CKD
    NPS = n // NS

    @functools.partial(
        pl.kernel,
        out_type=jax.ShapeDtypeStruct((NC, n, D), jnp.float32),
        mesh=plsc.VectorSubcoreMesh(core_axis_name="c", subcore_axis_name="s"),
        scratch_types=[pltpu.VMEM((1, CKD), jnp.int32),
                       pltpu.VMEM((1, CKD), jnp.int32),
                       pltpu.VMEM((CKD, D), jnp.float32),
                       pltpu.VMEM((CKD, D), jnp.float32),
                       pltpu.VMEM((CKD, D), jnp.float32),
                       pltpu.VMEM((Hh,), jnp.float32),
                       pltpu.VMEM_SHARED((n, D), jnp.float32),
                       pltpu.SemaphoreType.DMA,
                       pltpu.SemaphoreType.DMA],
    )
    def k(comb_hbm, ii2_hbm, cmax_hbm, init_hbm, out_hbm,
          ix0, ix1, bv0, bv1, py0, cm_v, acc_sh, rs0, rs1):
        ixv = (ix0, ix1)
        bv = (bv0, bv1)
        pay = (py0, py0)
        rs = (rs0, rs1)
        c = lax.axis_index("c")
        sid = lax.axis_index("s")
        pltpu.sync_copy(init_hbm.at[c, pl.ds(sid * NPS, NPS)],
                        acc_sh.at[pl.ds(sid * NPS, NPS)])
        pltpu.sync_copy(cmax_hbm.at[c], cm_v)
        plsc.subcore_barrier()
        cms = [cm_v[pl.ds(g * 16, 16)] for g in range(Hh // 16)]

        def fire(kk, b):
            e0 = sid * ESC + kk * CKD
            r0 = sid * NCHD + kk
            pltpu.async_copy(ii2_hbm.at[pl.ds(r0, 1)], ixv[b], rs[b])
            pltpu.async_copy(comb_hbm.at[c, pl.ds(e0, CKD)], bv[b], rs[b])

        def drain_reads(b):
            pltpu.make_async_copy(ii2_hbm.at[pl.ds(0, 1)], ixv[b], rs[b]).wait()
            pltpu.make_async_copy(comb_hbm.at[c, pl.ds(0, CKD)], bv[b],
                                  rs[b]).wait()

        def compute_and_scatter(b):
            def rowfn(r, cc):
                for rr in (2 * r, 2 * r + 1):
                    for g in range(Hh // 16):
                        ex = jnp.exp(bv[b][rr, pl.ds(g * 16, 16)] - cms[g])
                        pay[b][rr, pl.ds(Hh + g * 16, 16)] = ex
                        pay[b][rr, pl.ds(g * 16, 16)] = \
                            ex * bv[b][rr, pl.ds(Hh + g * 16, 16)]
                return cc

            lax.fori_loop(0, CKD // 2, rowfn, 0)
            pltpu.sync_copy(pay[b], acc_sh.at[ixv[b].at[0]], add=True)

        fire(0, 0)
        fire(1, 1)

        def body(g, cc):
            drain_reads(0)
            compute_and_scatter(0)
            nxt0 = 2 * g + 2

            @pl.when(nxt0 < NCHD)
            def _():
                fire(nxt0, 0)

            drain_reads(1)
            compute_and_scatter(1)
            nxt1 = 2 * g + 3

            @pl.when(nxt1 < NCHD)
            def _():
                fire(nxt1, 1)

            return cc

        lax.fori_loop(0, NCHD // 2, body, 0)
        if NCHD % 2 == 1:
            drain_reads(0)
            compute_and_scatter(0)
        plsc.subcore_barrier()
        pltpu.sync_copy(acc_sh.at[pl.ds(sid * NPS, NPS)],
                        out_hbm.at[c, pl.ds(sid * NPS, NPS)])

    return k(comb, ii2, cmax2, init)


def _stage_e_body(acc_ref, alphaL_ref, sL_ref, cmax_ref, W_out_ref, b_out_ref,
                  o_ref):
    D = o_ref.shape[1]
    Hh = D // 2
    exL = jnp.exp(alphaL_ref[...] - cmax_ref[...])
    sL = sL_ref[...]
    num0 = acc_ref[0, :, 0:Hh] + exL[:, 0:Hh] * sL[:, 0:Hh]
    den0 = acc_ref[0, :, Hh:D] + exL[:, 0:Hh]
    num1 = acc_ref[1, :, 0:Hh] + exL[:, Hh:D] * sL[:, Hh:D]
    den1 = acc_ref[1, :, Hh:D] + exL[:, Hh:D]
    o0 = num0 / (den0 + 1e-16)
    o1 = num1 / (den1 + 1e-16)
    out = (o0 @ W_out_ref[0:Hh, :] + o1 @ W_out_ref[Hh:D, :]) + b_out_ref[...]
    o_ref[...] = jnp.maximum(out, 0.0)


def _stage_e(accsc, alphaL, sL, cmax_row, W_out, b_out):
    n, D = alphaL.shape
    grid = (n // BN,)
    full = lambda shape: pl.BlockSpec(shape, lambda i: (0, 0))
    row = lambda w: pl.BlockSpec((BN, w), lambda i: (i, 0))
    return pl.pallas_call(
        _stage_e_body,
        grid=grid,
        in_specs=[pl.BlockSpec((NC, BN, D), lambda i: (0, i, 0)),
                  row(D), row(D), full((1, D)), full((D, D)), full((1, D))],
        out_specs=row(D),
        out_shape=jax.ShapeDtypeStruct((n, D), jnp.float32),
    )(accsc, alphaL, sL, cmax_row, W_out, b_out)


def kernel(x, pos, edge_index, W_in, b_in, W_out, b_out, W_lin, W_src, W_dst,
           Wp1, bp1, Wp2, bp2, Wa1, ba1, Wa2, ba2):
    n, D = x.shape
    E = edge_index.shape[1]
    jj = edge_index[0].astype(jnp.int32)
    ii = edge_index[1].astype(jnp.int32)
    pos8 = jnp.pad(pos.astype(jnp.float32), ((0, 0), (0, 8 - pos.shape[1])))
    Wp1_p = jnp.pad(Wp1, ((0, 8 - Wp1.shape[0]), (0, D - Wp1.shape[1])))
    r1 = lambda v: v.reshape(1, -1)

    tbl, sL, alphaL, cmaxA = _stage_a(
        n, x, pos8, W_in, r1(b_in), W_src, W_dst, W_lin, Wp1_p, r1(bp1), Wp2,
        r1(bp2), Wa1, r1(ba1), Wa2, r1(ba2))
    cmax = jnp.max(cmaxA, axis=0)
    npad = ((n + NS * 8 - 1) // (NS * 8)) * (NS * 8)
    accsc = jnp.zeros((NC, npad, D), jnp.float32)

    unit = 32 * CK
    nu = E // unit
    sl = [(nu // 4 + (1 if t < nu % 4 else 0)) * unit for t in range(4)]
    a0 = 0
    for Es in sl:
        iis = lax.dynamic_slice_in_dim(ii, a0, Es)
        jjs = lax.dynamic_slice_in_dim(jj, a0, Es)
        a0 += Es
        ijt = jnp.stack([iis.reshape(Es // CK, CK),
                         jjs.reshape(Es // CK, CK)], axis=1)
        hi, hj, qd = _gather_sc(tbl, ijt, Es)
        comb = _stage_c(
            hi, hj, qd, W_src, W_dst, W_lin, r1(bp1), Wp2, r1(bp2), Wa1,
            r1(ba1), Wa2, r1(ba2))
        accsc = _scatter_sc(comb, iis.reshape(Es // CKD, CKD),
                            cmax.reshape(NC, D // NC), accsc)
    return _stage_e(accsc, alphaL, sL, cmax.reshape(1, -1), W_out, r1(b_out))


# speedup vs baseline: 9.2185x; 1.0317x over previous
import functools

import jax
import jax.numpy as jnp
from jax import lax
from jax.experimental import pallas as pl
from jax.experimental.pallas import tpu as pltpu
from jax.experimental.pallas import tpu_sc as plsc

NC = 2
NS = 16
BN = 1000
KE = 1280
CK = 80
CKD = 40


def _stage_a_body(x_ref, pos_ref, W_in_ref, b_in_ref, W_src_ref, W_dst_ref,
                  W_lin_ref, Wp1_ref, bp1_ref, Wp2_ref, bp2_ref,
                  Wa1_ref, ba1_ref, Wa2_ref, ba2_ref,
                  t_ref, sL_ref, alphaL_ref, cmax_ref):
    pid = pl.program_id(0)
    D = x_ref.shape[1]
    h = jnp.maximum(x_ref[...] @ W_in_ref[...] + b_in_ref[...], 0.0)
    t_ref[:, 0:D] = h
    t_ref[:, D:2 * D] = pos_ref[...] @ Wp1_ref[...]
    d0 = jnp.maximum(jnp.maximum(bp1_ref[...], 0.0) @ Wp2_ref[...]
                     + bp2_ref[...], 0.0)
    sL_ref[...] = h @ W_lin_ref[...] + d0
    t = h @ W_dst_ref[...] - h @ W_src_ref[...] + d0
    a1 = jnp.maximum(t @ Wa1_ref[...] + ba1_ref[...], 0.0)
    aL = jnp.maximum(a1 @ Wa2_ref[...] + ba2_ref[...], 0.0)
    alphaL_ref[...] = aL

    @pl.when(pid == 0)
    def _():
        cmax_ref[...] = jnp.zeros_like(cmax_ref)

    cm = jnp.max(aL, axis=0, keepdims=True)
    cmax_ref[...] = jnp.maximum(cmax_ref[...], jnp.broadcast_to(cm, cmax_ref.shape))


def _stage_a(n, x, pos8, W_in, b_in, W_src, W_dst, W_lin, Wp1_p, bp1, Wp2, bp2,
             Wa1, ba1, Wa2, ba2):
    D = x.shape[1]
    H = bp1.shape[1]
    grid = (n // BN,)
    full = lambda shape: pl.BlockSpec(shape, lambda i: (0, 0))
    row = lambda w: pl.BlockSpec((BN, w), lambda i: (i, 0))
    return pl.pallas_call(
        _stage_a_body,
        grid=grid,
        in_specs=[row(D), pl.BlockSpec((BN, 8), lambda i: (i, 0)),
                  full((D, D)), full((1, D)), full((D, D)), full((D, D)),
                  full((D, D)), full((8, D)), full((1, H)), full((H, D)),
                  full((1, D)), full((D, H)), full((1, H)), full((H, D)),
                  full((1, D))],
        out_specs=[row(2 * D), row(D), row(D),
                   pl.BlockSpec((8, D), lambda i: (0, 0))],
        out_shape=[jax.ShapeDtypeStruct((n, 2 * D), jnp.float32),
                   jax.ShapeDtypeStruct((n, D), jnp.float32),
                   jax.ShapeDtypeStruct((n, D), jnp.float32),
                   jax.ShapeDtypeStruct((8, D), jnp.float32)],
    )(x, pos8, W_in, b_in, W_src, W_dst, W_lin, Wp1_p, bp1, Wp2, bp2,
      Wa1, ba1, Wa2, ba2)


def _gather_sc(tbl, ijt, E):
    n2, TW = tbl.shape
    D = TW // 2
    Hq = 64
    EW = E // (NC * NS)
    NCH = EW // CK

    @functools.partial(
        pl.kernel,
        out_type=[jax.ShapeDtypeStruct((E, D), jnp.float32),
                  jax.ShapeDtypeStruct((E, D), jnp.float32),
                  jax.ShapeDtypeStruct((E, Hq), jnp.float32)],
        mesh=plsc.VectorSubcoreMesh(core_axis_name="c", subcore_axis_name="s"),
        scratch_types=[pltpu.VMEM((2, CK), jnp.int32),
                       pltpu.VMEM((2, CK), jnp.int32),
                       pltpu.VMEM((CK, TW), jnp.float32),
                       pltpu.VMEM((CK, TW), jnp.float32),
                       pltpu.VMEM((CK, TW), jnp.float32),
                       pltpu.VMEM((CK, TW), jnp.float32),
                       pltpu.VMEM((CK, Hq), jnp.float32),
                       pltpu.VMEM((CK, Hq), jnp.float32),
                       pltpu.SemaphoreType.DMA,
                       pltpu.SemaphoreType.DMA,
                       pltpu.SemaphoreType.DMA,
                       pltpu.SemaphoreType.DMA],
    )
    def k(tbl_hbm, ijt_hbm, hi_hbm, hj_hbm, qd_hbm,
          ij0, ij1, bi0, bi1, bj0, bj1, qd0, qd1, gs0, gs1, ws0, ws1):
        ijv = (ij0, ij1)
        bi = (bi0, bi1)
        bj = (bj0, bj1)
        qdv = (qd0, qd1)
        gs = (gs0, gs1)
        ws = (ws0, ws1)
        wid = lax.axis_index("s") * NC + lax.axis_index("c")
        base = wid * EW
        bch = wid * NCH

        def fire(kk, b):
            pltpu.sync_copy(ijt_hbm.at[bch + kk], ijv[b])
            c1 = pltpu.async_copy(tbl_hbm.at[ijv[b].at[0]], bi[b], gs[b])
            c2 = pltpu.async_copy(tbl_hbm.at[ijv[b].at[1]], bj[b], gs[b])
            return c1, c2

        def emit(kk, b, c1, c2):
            c1.wait()
            c2.wait()

            def rowfn(r, cc):
                for rr in (2 * r, 2 * r + 1):
                    for g in range(Hq // 16):
                        o = pl.ds(D + g * 16, 16)
                        qdv[b][rr, pl.ds(g * 16, 16)] = bi[b][rr, o] - bj[b][rr, o]
                return cc

            lax.fori_loop(0, CK // 2, rowfn, 0)
            e0 = base + kk * CK
            pltpu.async_copy(bi[b].at[pl.ds(0, CK), pl.ds(0, D)],
                             hi_hbm.at[pl.ds(e0, CK)], ws[b])
            pltpu.async_copy(bj[b].at[pl.ds(0, CK), pl.ds(0, D)],
                             hj_hbm.at[pl.ds(e0, CK)], ws[b])
            pltpu.async_copy(qdv[b], qd_hbm.at[pl.ds(e0, CK)], ws[b])

        def drain_writes(b):
            pltpu.make_async_copy(bi[b].at[pl.ds(0, CK), pl.ds(0, D)],
                                  hi_hbm.at[pl.ds(0, CK)], ws[b]).wait()
            pltpu.make_async_copy(bj[b].at[pl.ds(0, CK), pl.ds(0, D)],
                                  hj_hbm.at[pl.ds(0, CK)], ws[b]).wait()
            pltpu.make_async_copy(qdv[b], qd_hbm.at[pl.ds(0, CK)], ws[b]).wait()

        c1, c2 = fire(0, 0)
        c3, c4 = fire(1, 1)
        emit(0, 0, c1, c2)
        emit(1, 1, c3, c4)

        def body(g, cc):
            drain_writes(0)
            a1, a2 = fire(2 * g, 0)
            drain_writes(1)
            b1, b2 = fire(2 * g + 1, 1)
            emit(2 * g, 0, a1, a2)
            emit(2 * g + 1, 1, b1, b2)
            return cc

        lax.fori_loop(1, NCH // 2, body, 0)
        if NCH % 2 == 1:
            drain_writes(0)
            e1, e2 = fire(NCH - 1, 0)
            emit(NCH - 1, 0, e1, e2)
        drain_writes(1)
        drain_writes(0)

    return k(tbl, ijt)


def _stage_c_body(hi_ref, hj_ref, qd_ref, W_src_ref, W_dst_ref,
                  W_lin_ref, bp1_ref, Wp2_ref, bp2_ref, Wa1_ref, ba1_ref,
                  Wa2_ref, ba2_ref, comb_ref):
    hi = hi_ref[...]
    hj = hj_ref[...]
    g1 = qd_ref[...] + bp1_ref[...]
    delta = jnp.maximum(jnp.maximum(g1, 0.0) @ Wp2_ref[...] + bp2_ref[...], 0.0)
    t = hi @ W_dst_ref[...] - hj @ W_src_ref[...] + delta
    a1 = jnp.maximum(t @ Wa1_ref[...] + ba1_ref[...], 0.0)
    alpha = jnp.maximum(a1 @ Wa2_ref[...] + ba2_ref[...], 0.0)
    s = hj @ W_lin_ref[...] + delta
    Dh = alpha.shape[1] // 2
    comb_ref[0] = jnp.concatenate([alpha[:, 0:Dh], s[:, 0:Dh]], axis=1)
    comb_ref[1] = jnp.concatenate([alpha[:, Dh:], s[:, Dh:]], axis=1)


def _stage_c(hi, hj, qd, W_src, W_dst, W_lin, bp1, Wp2, bp2, Wa1, ba1,
             Wa2, ba2):
    E, D = hi.shape
    H = bp1.shape[1]
    grid = (E // KE,)
    full = lambda shape: pl.BlockSpec(shape, lambda i: (0, 0))
    row = lambda w: pl.BlockSpec((KE, w), lambda i: (i, 0))
    return pl.pallas_call(
        _stage_c_body,
        grid=grid,
        in_specs=[row(D), row(D), row(H),
                  full((D, D)), full((D, D)), full((D, D)), full((1, H)),
                  full((H, D)), full((1, D)), full((D, H)), full((1, H)),
                  full((H, D)), full((1, D))],
        out_specs=pl.BlockSpec((NC, KE, D), lambda i: (0, i, 0)),
        out_shape=jax.ShapeDtypeStruct((NC, E, D), jnp.float32),
    )(hi, hj, qd, W_src, W_dst, W_lin, bp1, Wp2, bp2, Wa1, ba1, Wa2, ba2)


def _scatter_sc(comb, ii2, cmax2, init):
    _, E, D = comb.shape
    Hh = D // 2
    n = init.shape[1]
    ESC = E // NS
    NCHD = ESC // CKD
    NPS = n // NS

    @functools.partial(
        pl.kernel,
        out_type=jax.ShapeDtypeStruct((NC, n, D), jnp.float32),
        mesh=plsc.VectorSubcoreMesh(core_axis_name="c", subcore_axis_name="s"),
        scratch_types=[pltpu.VMEM((1, CKD), jnp.int32),
                       pltpu.VMEM((1, CKD), jnp.int32),
                       pltpu.VMEM((1, CKD), jnp.int32),
                       pltpu.VMEM((1, CKD), jnp.int32),
                       pltpu.VMEM((CKD, D), jnp.float32),
                       pltpu.VMEM((CKD, D), jnp.float32),
                       pltpu.VMEM((CKD, D), jnp.float32),
                       pltpu.VMEM((CKD, D), jnp.float32),
                       pltpu.VMEM((Hh,), jnp.float32),
                       pltpu.VMEM_SHARED((n, D), jnp.float32),
                       pltpu.SemaphoreType.DMA,
                       pltpu.SemaphoreType.DMA,
                       pltpu.SemaphoreType.DMA,
                       pltpu.SemaphoreType.DMA],
    )
    def k(comb_hbm, ii2_hbm, cmax_hbm, init_hbm, out_hbm,
          ix0, ix1, ix2, ix3, bv0, bv1, py0, py1, cm_v, acc_sh,
          rs0, rs1, ss0, ss1):
        ixv = (ix0, ix1, ix2, ix3)
        bv = (bv0, bv1)
        pay = (py0, py1)
        rs = (rs0, rs1)
        ss = (ss0, ss1)
        c = lax.axis_index("c")
        sid = lax.axis_index("s")
        pltpu.sync_copy(init_hbm.at[c, pl.ds(sid * NPS, NPS)],
                        acc_sh.at[pl.ds(sid * NPS, NPS)])
        pltpu.sync_copy(cmax_hbm.at[c], cm_v)
        plsc.subcore_barrier()
        cms = [cm_v[pl.ds(g * 16, 16)] for g in range(Hh // 16)]

        def fire(kk, s2, s4):
            e0 = sid * ESC + kk * CKD
            r0 = sid * NCHD + kk
            pltpu.async_copy(ii2_hbm.at[pl.ds(r0, 1)], ixv[s4], rs[s2])
            pltpu.async_copy(comb_hbm.at[c, pl.ds(e0, CKD)], bv[s2], rs[s2])

        def drain_reads(s2, s4):
            pltpu.make_async_copy(ii2_hbm.at[pl.ds(0, 1)], ixv[s4],
                                  rs[s2]).wait()
            pltpu.make_async_copy(comb_hbm.at[c, pl.ds(0, CKD)], bv[s2],
                                  rs[s2]).wait()

        def compute(s2):
            def rowfn(r, cc):
                for rr in (2 * r, 2 * r + 1):
                    for g in range(Hh // 16):
                        ex = jnp.exp(bv[s2][rr, pl.ds(g * 16, 16)] - cms[g])
                        pay[s2][rr, pl.ds(Hh + g * 16, 16)] = ex
                        pay[s2][rr, pl.ds(g * 16, 16)] = \
                            ex * bv[s2][rr, pl.ds(Hh + g * 16, 16)]
                return cc

            lax.fori_loop(0, CKD // 2, rowfn, 0)

        def fire_scatter(s2, s4):
            pltpu.async_copy(pay[s2], acc_sh.at[ixv[s4].at[0]], ss[s2],
                             add=True)

        def drain_scatter(s2):
            pltpu.make_async_copy(pay[s2], acc_sh.at[ixv[0].at[0]],
                                  ss[s2]).wait()

        def phase(kk, s2, s4, first):
            drain_reads(s2, s4)
            if not first:
                drain_scatter(s2)
            compute(s2)
            fire_scatter(s2, s4)
            nxt = kk + 2

            @pl.when(nxt < NCHD)
            def _():
                fire(nxt, s2, (s4 + 2) % 4)

        fire(0, 0, 0)
        fire(1, 1, 1)
        phase(0, 0, 0, True)
        phase(1, 1, 1, True)
        phase(2, 0, 2, False)
        phase(3, 1, 3, False)

        def body(g, cc):
            for j in range(4):
                phase(4 * g + j, j % 2, j, False)
            return cc

        lax.fori_loop(1, NCHD // 4, body, 0)
        drain_scatter(0)
        drain_scatter(1)
        plsc.subcore_barrier()
        pltpu.sync_copy(acc_sh.at[pl.ds(sid * NPS, NPS)],
                        out_hbm.at[c, pl.ds(sid * NPS, NPS)])

    return k(comb, ii2, cmax2, init)


def _stage_e_body(acc_ref, alphaL_ref, sL_ref, cmax_ref, W_out_ref, b_out_ref,
                  o_ref):
    D = o_ref.shape[1]
    Hh = D // 2
    exL = jnp.exp(alphaL_ref[...] - cmax_ref[...])
    sL = sL_ref[...]
    num0 = acc_ref[0, :, 0:Hh] + exL[:, 0:Hh] * sL[:, 0:Hh]
    den0 = acc_ref[0, :, Hh:D] + exL[:, 0:Hh]
    num1 = acc_ref[1, :, 0:Hh] + exL[:, Hh:D] * sL[:, Hh:D]
    den1 = acc_ref[1, :, Hh:D] + exL[:, Hh:D]
    o0 = num0 / (den0 + 1e-16)
    o1 = num1 / (den1 + 1e-16)
    out = (o0 @ W_out_ref[0:Hh, :] + o1 @ W_out_ref[Hh:D, :]) + b_out_ref[...]
    o_ref[...] = jnp.maximum(out, 0.0)


def _stage_e(accsc, alphaL, sL, cmax_row, W_out, b_out):
    n, D = alphaL.shape
    grid = (n // BN,)
    full = lambda shape: pl.BlockSpec(shape, lambda i: (0, 0))
    row = lambda w: pl.BlockSpec((BN, w), lambda i: (i, 0))
    return pl.pallas_call(
        _stage_e_body,
        grid=grid,
        in_specs=[pl.BlockSpec((NC, BN, D), lambda i: (0, i, 0)),
                  row(D), row(D), full((1, D)), full((D, D)), full((1, D))],
        out_specs=row(D),
        out_shape=jax.ShapeDtypeStruct((n, D), jnp.float32),
    )(accsc, alphaL, sL, cmax_row, W_out, b_out)


def kernel(x, pos, edge_index, W_in, b_in, W_out, b_out, W_lin, W_src, W_dst,
           Wp1, bp1, Wp2, bp2, Wa1, ba1, Wa2, ba2):
    n, D = x.shape
    E = edge_index.shape[1]
    jj = edge_index[0].astype(jnp.int32)
    ii = edge_index[1].astype(jnp.int32)
    pos8 = jnp.pad(pos.astype(jnp.float32), ((0, 0), (0, 8 - pos.shape[1])))
    Wp1_p = jnp.pad(Wp1, ((0, 8 - Wp1.shape[0]), (0, D - Wp1.shape[1])))
    r1 = lambda v: v.reshape(1, -1)

    tbl, sL, alphaL, cmaxA = _stage_a(
        n, x, pos8, W_in, r1(b_in), W_src, W_dst, W_lin, Wp1_p, r1(bp1), Wp2,
        r1(bp2), Wa1, r1(ba1), Wa2, r1(ba2))
    cmax = jnp.max(cmaxA, axis=0)
    npad = ((n + NS * 8 - 1) // (NS * 8)) * (NS * 8)
    accsc = jnp.zeros((NC, npad, D), jnp.float32)

    unit = 32 * CK
    nu = E // unit
    sl = [(nu // 4 + (1 if t < nu % 4 else 0)) * unit for t in range(4)]
    a0 = 0
    for Es in sl:
        iis = lax.dynamic_slice_in_dim(ii, a0, Es)
        jjs = lax.dynamic_slice_in_dim(jj, a0, Es)
        a0 += Es
        ijt = jnp.stack([iis.reshape(Es // CK, CK),
                         jjs.reshape(Es // CK, CK)], axis=1)
        hi, hj, qd = _gather_sc(tbl, ijt, Es)
        comb = _stage_c(
            hi, hj, qd, W_src, W_dst, W_lin, r1(bp1), Wp2, r1(bp2), Wa1,
            r1(ba1), Wa2, r1(ba2))
        accsc = _scatter_sc(comb, iis.reshape(Es // CKD, CKD),
                            cmax.reshape(NC, D // NC), accsc)
    return _stage_e(accsc, alphaL, sL, cmax.reshape(1, -1), W_out, r1(b_out))


# speedup vs baseline: 9.2852x; 1.0072x over previous
import functools

import jax
import jax.numpy as jnp
from jax import lax
from jax.experimental import pallas as pl
from jax.experimental.pallas import tpu as pltpu
from jax.experimental.pallas import tpu_sc as plsc

NC = 2
NS = 16
BN = 1000
KE = 1280
CK = 80
CKD = 40


def _stage_a_body(x_ref, pos_ref, W_in_ref, b_in_ref, W_src_ref, W_dst_ref,
                  W_lin_ref, Wp1_ref, bp1_ref, Wp2_ref, bp2_ref,
                  Wa1_ref, ba1_ref, Wa2_ref, ba2_ref,
                  t_ref, sL_ref, alphaL_ref, cmax_ref):
    pid = pl.program_id(0)
    D = x_ref.shape[1]
    h = jnp.maximum(x_ref[...] @ W_in_ref[...] + b_in_ref[...], 0.0)
    t_ref[:, 0:D] = h
    t_ref[:, D:2 * D] = pos_ref[...] @ Wp1_ref[...]
    d0 = jnp.maximum(jnp.maximum(bp1_ref[...], 0.0) @ Wp2_ref[...]
                     + bp2_ref[...], 0.0)
    sL_ref[...] = h @ W_lin_ref[...] + d0
    t = h @ W_dst_ref[...] - h @ W_src_ref[...] + d0
    a1 = jnp.maximum(t @ Wa1_ref[...] + ba1_ref[...], 0.0)
    aL = jnp.maximum(a1 @ Wa2_ref[...] + ba2_ref[...], 0.0)
    alphaL_ref[...] = aL

    @pl.when(pid == 0)
    def _():
        cmax_ref[...] = jnp.zeros_like(cmax_ref)

    cm = jnp.max(aL, axis=0, keepdims=True)
    cmax_ref[...] = jnp.maximum(cmax_ref[...], jnp.broadcast_to(cm, cmax_ref.shape))


def _stage_a(n, x, pos8, W_in, b_in, W_src, W_dst, W_lin, Wp1_p, bp1, Wp2, bp2,
             Wa1, ba1, Wa2, ba2):
    D = x.shape[1]
    H = bp1.shape[1]
    grid = (n // BN,)
    full = lambda shape: pl.BlockSpec(shape, lambda i: (0, 0))
    row = lambda w: pl.BlockSpec((BN, w), lambda i: (i, 0))
    return pl.pallas_call(
        _stage_a_body,
        grid=grid,
        in_specs=[row(D), pl.BlockSpec((BN, 8), lambda i: (i, 0)),
                  full((D, D)), full((1, D)), full((D, D)), full((D, D)),
                  full((D, D)), full((8, D)), full((1, H)), full((H, D)),
                  full((1, D)), full((D, H)), full((1, H)), full((H, D)),
                  full((1, D))],
        out_specs=[row(2 * D), row(D), row(D),
                   pl.BlockSpec((8, D), lambda i: (0, 0))],
        out_shape=[jax.ShapeDtypeStruct((n, 2 * D), jnp.float32),
                   jax.ShapeDtypeStruct((n, D), jnp.float32),
                   jax.ShapeDtypeStruct((n, D), jnp.float32),
                   jax.ShapeDtypeStruct((8, D), jnp.float32)],
    )(x, pos8, W_in, b_in, W_src, W_dst, W_lin, Wp1_p, bp1, Wp2, bp2,
      Wa1, ba1, Wa2, ba2)


def _gather_sc(tbl, ijt, E):
    n2, TW = tbl.shape
    D = TW // 2
    Hq = 64
    EW = E // (NC * NS)
    NCH = EW // CK

    @functools.partial(
        pl.kernel,
        out_type=[jax.ShapeDtypeStruct((E, D), jnp.float32),
                  jax.ShapeDtypeStruct((E, D), jnp.float32),
                  jax.ShapeDtypeStruct((E, Hq), jnp.float32)],
        mesh=plsc.VectorSubcoreMesh(core_axis_name="c", subcore_axis_name="s"),
        scratch_types=[pltpu.VMEM((2, CK), jnp.int32),
                       pltpu.VMEM((2, CK), jnp.int32),
                       pltpu.VMEM((CK, TW), jnp.float32),
                       pltpu.VMEM((CK, TW), jnp.float32),
                       pltpu.VMEM((CK, TW), jnp.float32),
                       pltpu.VMEM((CK, TW), jnp.float32),
                       pltpu.VMEM((CK, Hq), jnp.float32),
                       pltpu.VMEM((CK, Hq), jnp.float32),
                       pltpu.SemaphoreType.DMA,
                       pltpu.SemaphoreType.DMA,
                       pltpu.SemaphoreType.DMA,
                       pltpu.SemaphoreType.DMA],
    )
    def k(tbl_hbm, ijt_hbm, hi_hbm, hj_hbm, qd_hbm,
          ij0, ij1, bi0, bi1, bj0, bj1, qd0, qd1, gs0, gs1, ws0, ws1):
        ijv = (ij0, ij1)
        bi = (bi0, bi1)
        bj = (bj0, bj1)
        qdv = (qd0, qd1)
        gs = (gs0, gs1)
        ws = (ws0, ws1)
        wid = lax.axis_index("s") * NC + lax.axis_index("c")
        base = wid * EW
        bch = wid * NCH

        def fire(kk, b):
            pltpu.sync_copy(ijt_hbm.at[bch + kk], ijv[b])
            c1 = pltpu.async_copy(tbl_hbm.at[ijv[b].at[0]], bi[b], gs[b])
            c2 = pltpu.async_copy(tbl_hbm.at[ijv[b].at[1]], bj[b], gs[b])
            return c1, c2

        def emit(kk, b, c1, c2):
            c1.wait()
            c2.wait()

            def rowfn(r, cc):
                for rr in (2 * r, 2 * r + 1):
                    for g in range(Hq // 16):
                        o = pl.ds(D + g * 16, 16)
                        qdv[b][rr, pl.ds(g * 16, 16)] = bi[b][rr, o] - bj[b][rr, o]
                return cc

            lax.fori_loop(0, CK // 2, rowfn, 0)
            e0 = base + kk * CK
            pltpu.async_copy(bi[b].at[pl.ds(0, CK), pl.ds(0, D)],
                             hi_hbm.at[pl.ds(e0, CK)], ws[b])
            pltpu.async_copy(bj[b].at[pl.ds(0, CK), pl.ds(0, D)],
                             hj_hbm.at[pl.ds(e0, CK)], ws[b])
            pltpu.async_copy(qdv[b], qd_hbm.at[pl.ds(e0, CK)], ws[b])

        def drain_writes(b):
            pltpu.make_async_copy(bi[b].at[pl.ds(0, CK), pl.ds(0, D)],
                                  hi_hbm.at[pl.ds(0, CK)], ws[b]).wait()
            pltpu.make_async_copy(bj[b].at[pl.ds(0, CK), pl.ds(0, D)],
                                  hj_hbm.at[pl.ds(0, CK)], ws[b]).wait()
            pltpu.make_async_copy(qdv[b], qd_hbm.at[pl.ds(0, CK)], ws[b]).wait()

        c1, c2 = fire(0, 0)
        c3, c4 = fire(1, 1)
        emit(0, 0, c1, c2)
        emit(1, 1, c3, c4)

        def body(g, cc):
            drain_writes(0)
            a1, a2 = fire(2 * g, 0)
            drain_writes(1)
            b1, b2 = fire(2 * g + 1, 1)
            emit(2 * g, 0, a1, a2)
            emit(2 * g + 1, 1, b1, b2)
            return cc

        lax.fori_loop(1, NCH // 2, body, 0)
        if NCH % 2 == 1:
            drain_writes(0)
            e1, e2 = fire(NCH - 1, 0)
            emit(NCH - 1, 0, e1, e2)
        drain_writes(1)
        drain_writes(0)

    return k(tbl, ijt)


def _stage_c_body(hi_ref, hj_ref, qd_ref, W_src_ref, W_dst_ref,
                  W_lin_ref, bp1_ref, Wp2_ref, bp2_ref, Wa1_ref, ba1_ref,
                  Wa2_ref, ba2_ref, comb_ref):
    hi = hi_ref[...]
    hj = hj_ref[...]
    g1 = qd_ref[...] + bp1_ref[...]
    delta = jnp.maximum(jnp.maximum(g1, 0.0) @ Wp2_ref[...] + bp2_ref[...], 0.0)
    t = hi @ W_dst_ref[...] - hj @ W_src_ref[...] + delta
    a1 = jnp.maximum(t @ Wa1_ref[...] + ba1_ref[...], 0.0)
    alpha = jnp.maximum(a1 @ Wa2_ref[...] + ba2_ref[...], 0.0)
    s = hj @ W_lin_ref[...] + delta
    Dh = alpha.shape[1] // 2
    comb_ref[0] = jnp.concatenate([alpha[:, 0:Dh], s[:, 0:Dh]], axis=1)
    comb_ref[1] = jnp.concatenate([alpha[:, Dh:], s[:, Dh:]], axis=1)


def _stage_c(hi, hj, qd, W_src, W_dst, W_lin, bp1, Wp2, bp2, Wa1, ba1,
             Wa2, ba2):
    E, D = hi.shape
    H = bp1.shape[1]
    grid = (E // KE,)
    full = lambda shape: pl.BlockSpec(shape, lambda i: (0, 0))
    row = lambda w: pl.BlockSpec((KE, w), lambda i: (i, 0))
    return pl.pallas_call(
        _stage_c_body,
        grid=grid,
        in_specs=[row(D), row(D), row(H),
                  full((D, D)), full((D, D)), full((D, D)), full((1, H)),
                  full((H, D)), full((1, D)), full((D, H)), full((1, H)),
                  full((H, D)), full((1, D))],
        out_specs=pl.BlockSpec((NC, KE, D), lambda i: (0, i, 0)),
        out_shape=jax.ShapeDtypeStruct((NC, E, D), jnp.float32),
    )(hi, hj, qd, W_src, W_dst, W_lin, bp1, Wp2, bp2, Wa1, ba1, Wa2, ba2)


def _scatter_sc(comb, ii2, cmax2, init):
    _, E, D = comb.shape
    Hh = D // 2
    n = init.shape[1]
    ESC = E // NS
    NCHD = ESC // CKD
    NPS = n // NS

    @functools.partial(
        pl.kernel,
        out_type=jax.ShapeDtypeStruct((NC, n, D), jnp.float32),
        mesh=plsc.VectorSubcoreMesh(core_axis_name="c", subcore_axis_name="s"),
        scratch_types=[pltpu.VMEM((1, CKD), jnp.int32),
                       pltpu.VMEM((1, CKD), jnp.int32),
                       pltpu.VMEM((1, CKD), jnp.int32),
                       pltpu.VMEM((1, CKD), jnp.int32),
                       pltpu.VMEM((CKD, D), jnp.float32),
                       pltpu.VMEM((CKD, D), jnp.float32),
                       pltpu.VMEM((CKD, D), jnp.float32),
                       pltpu.VMEM((CKD, D), jnp.float32),
                       pltpu.VMEM((Hh,), jnp.float32),
                       pltpu.VMEM_SHARED((n, D), jnp.float32),
                       pltpu.SemaphoreType.DMA,
                       pltpu.SemaphoreType.DMA,
                       pltpu.SemaphoreType.DMA,
                       pltpu.SemaphoreType.DMA],
    )
    def k(comb_hbm, ii2_hbm, cmax_hbm, init_hbm, out_hbm,
          ix0, ix1, ix2, ix3, bv0, bv1, py0, py1, cm_v, acc_sh,
          rs0, rs1, ss0, ss1):
        ixv = (ix0, ix1, ix2, ix3)
        bv = (bv0, bv1)
        pay = (py0, py1)
        rs = (rs0, rs1)
        ss = (ss0, ss1)
        c = lax.axis_index("c")
        sid = lax.axis_index("s")
        pltpu.sync_copy(init_hbm.at[c, pl.ds(sid * NPS, NPS)],
                        acc_sh.at[pl.ds(sid * NPS, NPS)])
        pltpu.sync_copy(cmax_hbm.at[c], cm_v)
        plsc.subcore_barrier()
        cms = [cm_v[pl.ds(g * 16, 16)] for g in range(Hh // 16)]

        def fire(kk, s2, s4):
            e0 = sid * ESC + kk * CKD
            r0 = sid * NCHD + kk
            pltpu.async_copy(ii2_hbm.at[pl.ds(r0, 1)], ixv[s4], rs[s2])
            pltpu.async_copy(comb_hbm.at[c, pl.ds(e0, CKD)], bv[s2], rs[s2])

        def drain_reads(s2, s4):
            pltpu.make_async_copy(ii2_hbm.at[pl.ds(0, 1)], ixv[s4],
                                  rs[s2]).wait()
            pltpu.make_async_copy(comb_hbm.at[c, pl.ds(0, CKD)], bv[s2],
                                  rs[s2]).wait()

        def compute(s2):
            def rowfn(r, cc):
                for rr in (2 * r, 2 * r + 1):
                    for g in range(Hh // 16):
                        ex = jnp.exp(bv[s2][rr, pl.ds(g * 16, 16)] - cms[g])
                        pay[s2][rr, pl.ds(Hh + g * 16, 16)] = ex
                        pay[s2][rr, pl.ds(g * 16, 16)] = \
                            ex * bv[s2][rr, pl.ds(Hh + g * 16, 16)]
                return cc

            lax.fori_loop(0, CKD // 2, rowfn, 0)

        def fire_scatter(s2, s4):
            pltpu.async_copy(pay[s2], acc_sh.at[ixv[s4].at[0]], ss[s2],
                             add=True)

        def drain_scatter(s2):
            pltpu.make_async_copy(pay[s2], acc_sh.at[ixv[0].at[0]],
                                  ss[s2]).wait()

        def phase(kk, s2, s4, first):
            drain_reads(s2, s4)
            if not first:
                drain_scatter(s2)
            compute(s2)
            fire_scatter(s2, s4)
            nxt = kk + 2

            @pl.when(nxt < NCHD)
            def _():
                fire(nxt, s2, (s4 + 2) % 4)

        fire(0, 0, 0)
        fire(1, 1, 1)
        phase(0, 0, 0, True)
        phase(1, 1, 1, True)
        phase(2, 0, 2, False)
        phase(3, 1, 3, False)

        def body(g, cc):
            for j in range(4):
                phase(4 * g + j, j % 2, j, False)
            return cc

        lax.fori_loop(1, NCHD // 4, body, 0)
        drain_scatter(0)
        drain_scatter(1)
        plsc.subcore_barrier()
        pltpu.sync_copy(acc_sh.at[pl.ds(sid * NPS, NPS)],
                        out_hbm.at[c, pl.ds(sid * NPS, NPS)])

    return k(comb, ii2, cmax2, init)


def _stage_e_body(acc_ref, alphaL_ref, sL_ref, cmax_ref, W_out_ref, b_out_ref,
                  o_ref):
    D = o_ref.shape[1]
    Hh = D // 2
    exL = jnp.exp(alphaL_ref[...] - cmax_ref[...])
    sL = sL_ref[...]
    num0 = acc_ref[0, :, 0:Hh] + exL[:, 0:Hh] * sL[:, 0:Hh]
    den0 = acc_ref[0, :, Hh:D] + exL[:, 0:Hh]
    num1 = acc_ref[1, :, 0:Hh] + exL[:, Hh:D] * sL[:, Hh:D]
    den1 = acc_ref[1, :, Hh:D] + exL[:, Hh:D]
    o0 = num0 / (den0 + 1e-16)
    o1 = num1 / (den1 + 1e-16)
    out = (o0 @ W_out_ref[0:Hh, :] + o1 @ W_out_ref[Hh:D, :]) + b_out_ref[...]
    o_ref[...] = jnp.maximum(out, 0.0)


def _stage_e(accsc, alphaL, sL, cmax_row, W_out, b_out):
    n, D = alphaL.shape
    grid = (n // BN,)
    full = lambda shape: pl.BlockSpec(shape, lambda i: (0, 0))
    row = lambda w: pl.BlockSpec((BN, w), lambda i: (i, 0))
    return pl.pallas_call(
        _stage_e_body,
        grid=grid,
        in_specs=[pl.BlockSpec((NC, BN, D), lambda i: (0, i, 0)),
                  row(D), row(D), full((1, D)), full((D, D)), full((1, D))],
        out_specs=row(D),
        out_shape=jax.ShapeDtypeStruct((n, D), jnp.float32),
    )(accsc, alphaL, sL, cmax_row, W_out, b_out)


def kernel(x, pos, edge_index, W_in, b_in, W_out, b_out, W_lin, W_src, W_dst,
           Wp1, bp1, Wp2, bp2, Wa1, ba1, Wa2, ba2):
    n, D = x.shape
    E = edge_index.shape[1]
    jj = edge_index[0].astype(jnp.int32)
    ii = edge_index[1].astype(jnp.int32)
    pos8 = jnp.pad(pos.astype(jnp.float32), ((0, 0), (0, 8 - pos.shape[1])))
    Wp1_p = jnp.pad(Wp1, ((0, 8 - Wp1.shape[0]), (0, D - Wp1.shape[1])))
    r1 = lambda v: v.reshape(1, -1)

    tbl, sL, alphaL, cmaxA = _stage_a(
        n, x, pos8, W_in, r1(b_in), W_src, W_dst, W_lin, Wp1_p, r1(bp1), Wp2,
        r1(bp2), Wa1, r1(ba1), Wa2, r1(ba2))
    cmax = jnp.max(cmaxA, axis=0)
    npad = ((n + NS * 8 - 1) // (NS * 8)) * (NS * 8)
    accsc = jnp.zeros((NC, npad, D), jnp.float32)

    unit = 32 * CK
    nu = E // unit
    u0 = max(1, (nu * 16) // 100)
    rest = nu - u0
    sl = [u0 * unit] + [(rest // 3 + (1 if t < rest % 3 else 0)) * unit
                        for t in range(3)]
    a0 = 0
    for Es in sl:
        iis = lax.dynamic_slice_in_dim(ii, a0, Es)
        jjs = lax.dynamic_slice_in_dim(jj, a0, Es)
        a0 += Es
        ijt = jnp.stack([iis.reshape(Es // CK, CK),
                         jjs.reshape(Es // CK, CK)], axis=1)
        hi, hj, qd = _gather_sc(tbl, ijt, Es)
        comb = _stage_c(
            hi, hj, qd, W_src, W_dst, W_lin, r1(bp1), Wp2, r1(bp2), Wa1,
            r1(ba1), Wa2, r1(ba2))
        accsc = _scatter_sc(comb, iis.reshape(Es // CKD, CKD),
                            cmax.reshape(NC, D // NC), accsc)
    return _stage_e(accsc, alphaL, sL, cmax.reshape(1, -1), W_out, r1(b_out))
